# Initial kernel scaffold; baseline (speedup 1.0000x reference)
import jax
import jax.numpy as jnp
from jax.experimental import pallas as pl


def kernel(x, pos, edge_index, Wh1, bh1, Wh2, bh2, Wf1, bf1, Wf2, bf2, Wg1, bg1, Wg2, bg2):
    raise NotImplementedError("write your pallas kernel here")


# speedup vs baseline: 1.5450x; 1.5450x over previous
import functools
import jax
import jax.numpy as jnp
from jax.experimental import pallas as pl

NEG = -jnp.inf


def _node_pre_body(x_ref, pos8_ref, Wh1_ref, bh1_ref, Wh2_ref, bh2_ref,
                   WfX_ref, W1r_ref, bf1_ref, A_ref, B_ref):
    x = x_ref[...]
    h = jnp.maximum(jnp.dot(x, Wh1_ref[...],
                            preferred_element_type=jnp.float32) + bh1_ref[...], 0.0)
    delta8 = jnp.dot(h, Wh2_ref[...], preferred_element_type=jnp.float32) + bh2_ref[...]
    q8 = delta8 - pos8_ref[...]
    B_ref[...] = jnp.dot(q8, W1r_ref[...], preferred_element_type=jnp.float32)
    A_ref[...] = (jnp.dot(x, WfX_ref[...], preferred_element_type=jnp.float32)
                  + jnp.dot(pos8_ref[...], W1r_ref[...], preferred_element_type=jnp.float32)
                  + bf1_ref[...])


def _edge_mlp_body(G_ref, Wf2_ref, bf2_ref, e_ref):
    h = jnp.maximum(G_ref[...], 0.0)
    e_ref[...] = jnp.dot(h, Wf2_ref[...], preferred_element_type=jnp.float32) + bf2_ref[...]


def _out_mlp_body(aggr_ref, x_ref, Wg1_ref, bg1_ref, Wg2_ref, bg2_ref, o_ref):
    a = aggr_ref[...]
    a = jnp.where(a == NEG, 0.0, a)
    h = jnp.maximum(jnp.dot(a, Wg1_ref[...],
                            preferred_element_type=jnp.float32) + bg1_ref[...], 0.0)
    o_ref[...] = (jnp.dot(h, Wg2_ref[...], preferred_element_type=jnp.float32)
                  + bg2_ref[...] + x_ref[...])


def kernel(x, pos, edge_index, Wh1, bh1, Wh2, bh2, Wf1, bf1, Wf2, bf2,
           Wg1, bg1, Wg2, bg2):
    N, D = x.shape
    E = edge_index.shape[1]
    NP = ((N + 15) // 16) * 16

    xp = jnp.pad(x, ((0, NP - N), (0, 0)))
    pos8 = jnp.pad(pos, ((0, NP - N), (0, 5)))
    Wh2p = jnp.pad(Wh2, ((0, 0), (0, 5)))
    bh2p = jnp.pad(bh2, ((0, 5))).reshape(1, 8)
    W1r = jnp.pad(Wf1[:3], ((0, 5), (0, 0)))
    WfX = Wf1[3:]

    BN = 1024
    gn = (NP + BN - 1) // BN
    A, B = pl.pallas_call(
        _node_pre_body,
        grid=(gn,),
        in_specs=[
            pl.BlockSpec((BN, D), lambda i: (i, 0)),
            pl.BlockSpec((BN, 8), lambda i: (i, 0)),
            pl.BlockSpec((D, 64), lambda i: (0, 0)),
            pl.BlockSpec((1, 64), lambda i: (0, 0)),
            pl.BlockSpec((64, 8), lambda i: (0, 0)),
            pl.BlockSpec((1, 8), lambda i: (0, 0)),
            pl.BlockSpec((D, 128), lambda i: (0, 0)),
            pl.BlockSpec((8, 128), lambda i: (0, 0)),
            pl.BlockSpec((1, 128), lambda i: (0, 0)),
        ],
        out_specs=[
            pl.BlockSpec((BN, 128), lambda i: (i, 0)),
            pl.BlockSpec((BN, 128), lambda i: (i, 0)),
        ],
        out_shape=[
            jax.ShapeDtypeStruct((NP, 128), jnp.float32),
            jax.ShapeDtypeStruct((NP, 128), jnp.float32),
        ],
    )(xp, pos8, Wh1, bh1.reshape(1, 64), Wh2p, bh2p, WfX, W1r,
      bf1.reshape(1, 128))

    src = edge_index[0]
    dst = edge_index[1]

    G = A[src] + B[dst]

    BE = 2000
    ge = (E + BE - 1) // BE
    e = pl.pallas_call(
        _edge_mlp_body,
        grid=(ge,),
        in_specs=[
            pl.BlockSpec((BE, 128), lambda i: (i, 0)),
            pl.BlockSpec((128, 128), lambda i: (0, 0)),
            pl.BlockSpec((1, 128), lambda i: (0, 0)),
        ],
        out_specs=pl.BlockSpec((BE, 128), lambda i: (i, 0)),
        out_shape=jax.ShapeDtypeStruct((E, 128), jnp.float32),
    )(G, Wf2, bf2.reshape(1, 128))

    aggr = jax.ops.segment_max(e, dst, num_segments=NP)
    aggr = jnp.where(jnp.isneginf(aggr), NEG, aggr)

    out = pl.pallas_call(
        _out_mlp_body,
        grid=(gn,),
        in_specs=[
            pl.BlockSpec((BN, 128), lambda i: (i, 0)),
            pl.BlockSpec((BN, D), lambda i: (i, 0)),
            pl.BlockSpec((128, 128), lambda i: (0, 0)),
            pl.BlockSpec((1, 128), lambda i: (0, 0)),
            pl.BlockSpec((128, D), lambda i: (0, 0)),
            pl.BlockSpec((1, D), lambda i: (0, 0)),
        ],
        out_specs=pl.BlockSpec((BN, D), lambda i: (i, 0)),
        out_shape=jax.ShapeDtypeStruct((NP, D), jnp.float32),
    )(aggr, xp, Wg1, bg1.reshape(1, 128), Wg2, bg2.reshape(1, D))

    return out[:N]


# speedup vs baseline: 1.5612x; 1.0105x over previous
import jax
import jax.numpy as jnp
from jax import lax
from jax.experimental import pallas as pl
from jax.experimental.pallas import tpu as pltpu
from jax.experimental.pallas import tpu_sc as plsc

NEG = -jnp.inf

_SC_INFO = plsc.get_sparse_core_info()
_NC, _NS = _SC_INFO.num_cores, _SC_INFO.num_subcores
_NW = _NC * _NS

_DIV_MUL = 52429


def _node_pre_body(x_ref, pos8_ref, Wh1_ref, bh1_ref, Wh2_ref, bh2_ref,
                   WfX_ref, W1r_ref, bf1_ref, A_ref, B_ref):
    x = x_ref[...]
    h = jnp.maximum(jnp.dot(x, Wh1_ref[...],
                            preferred_element_type=jnp.float32) + bh1_ref[...], 0.0)
    delta8 = jnp.dot(h, Wh2_ref[...], preferred_element_type=jnp.float32) + bh2_ref[...]
    q8 = delta8 - pos8_ref[...]
    B_ref[...] = jnp.dot(q8, W1r_ref[...], preferred_element_type=jnp.float32)
    A_ref[...] = (jnp.dot(x, WfX_ref[...], preferred_element_type=jnp.float32)
                  + jnp.dot(pos8_ref[...], W1r_ref[...], preferred_element_type=jnp.float32)
                  + bf1_ref[...])


def _k1_body(dstm_ref, cc_ref):
    i = pl.program_id(0)

    @pl.when(i == 0)
    def _():
        cc_ref[...] = jnp.zeros_like(cc_ref)

    b = lax.shift_right_logical(dstm_ref[...] * _DIV_MUL, 24)
    for B in range(_NW):
        eq = (b == B).astype(jnp.float32)
        cc_ref[pl.ds(B, 1), :] += jnp.sum(eq, axis=0, keepdims=True)


def _k2_body(cc_ref, U128_ref, L32_ref, lanebase_ref, bb2_ref):
    cc = cc_ref[...]
    laneprefix = jnp.dot(cc, U128_ref[...], preferred_element_type=jnp.float32)
    totals = jnp.sum(cc, axis=1, keepdims=True)
    ru = jnp.floor((totals + 127.0) * (1.0 / 128.0)) * 128.0
    ab = jnp.dot(L32_ref[...], ru, preferred_element_type=jnp.float32)
    lanebase_ref[...] = ab + laneprefix
    ends = ab + totals
    bb2_ref[...] = jnp.concatenate(
        [ab, ends, jnp.zeros((_NW, 14), jnp.float32)], axis=1).astype(jnp.int32)


def _k3_body(dstm_ref, lanebase_ref, L_ref, slot_ref, carry_ref):
    i = pl.program_id(0)

    @pl.when(i == 0)
    def _():
        carry_ref[...] = jnp.zeros_like(carry_ref)

    b = lax.shift_right_logical(dstm_ref[...] * _DIV_MUL, 24)
    acc = jnp.zeros(slot_ref.shape, jnp.float32)
    L = L_ref[...]
    for B in range(_NW):
        eq = (b == B).astype(jnp.float32)
        p = jnp.dot(L, eq, preferred_element_type=jnp.float32)
        base = carry_ref[pl.ds(B, 1), :] + lanebase_ref[pl.ds(B, 1), :]
        acc = acc + eq * (p + base)
        carry_ref[pl.ds(B, 1), :] += jnp.sum(eq, axis=0, keepdims=True)
    slot_ref[...] = acc.astype(jnp.int32)


def _edge_mlp_body(G_ref, Wf2_ref, bf2_ref, e_ref):
    h = jnp.maximum(G_ref[...], 0.0)
    e_ref[...] = jnp.dot(h, Wf2_ref[...], preferred_element_type=jnp.float32) + bf2_ref[...]


def _out_mlp_body(aggr_ref, x_ref, Wg1_ref, bg1_ref, Wg2_ref, bg2_ref, o_ref):
    a = aggr_ref[...]
    a = jnp.where(a == NEG, 0.0, a)
    h = jnp.maximum(jnp.dot(a, Wg1_ref[...],
                            preferred_element_type=jnp.float32) + bg1_ref[...], 0.0)
    o_ref[...] = (jnp.dot(h, Wg2_ref[...], preferred_element_type=jnp.float32)
                  + bg2_ref[...] + x_ref[...])


def _scatter_body(src_hbm, dst_hbm, slot_hbm, idp_hbm, srcp_hbm, dstp_hbm,
                  ramp, idbuf, srcbuf, dstbuf, slotbuf):
    E = src_hbm.shape[0]
    CH = 128
    nch_total = E // CH
    wid = lax.axis_index("s") * _NC + lax.axis_index("c")
    n_my = (nch_total - wid + _NW - 1) // _NW

    iota = lax.iota(jnp.int32, 16)
    for c in range(CH // 16):
        ramp[pl.ds(c * 16, 16)] = iota + (c * 16)

    def ch_body(ch, _):
        base = pl.multiple_of((wid + ch * _NW) * CH, CH)
        pltpu.sync_copy(src_hbm.at[pl.ds(base, CH)], srcbuf)
        pltpu.sync_copy(dst_hbm.at[pl.ds(base, CH)], dstbuf)
        pltpu.sync_copy(slot_hbm.at[pl.ds(base, CH)], slotbuf)
        for c in range(CH // 16):
            sl = pl.ds(c * 16, 16)
            idbuf[sl] = ramp[sl] + base
        pltpu.sync_copy(idbuf, idp_hbm.at[slotbuf])
        pltpu.sync_copy(srcbuf, srcp_hbm.at[slotbuf])
        pltpu.sync_copy(dstbuf, dstp_hbm.at[slotbuf])
        return 0

    lax.fori_loop(0, n_my, ch_body, 0)


def _gather_add_body(srcp_hbm, dstp_hbm, A_hbm, B_hbm, G_hbm,
                     sidx, didx, rowsA, rowsB, semA, semB):
    EP = srcp_hbm.shape[0]
    NPAD = A_hbm.shape[0]
    CH = 128
    nch_total = EP // CH
    wid = lax.axis_index("s") * _NC + lax.axis_index("c")
    n_my = (nch_total - wid + _NW - 1) // _NW

    def ch_body(i, _):
        base = pl.multiple_of((wid + i * _NW) * CH, CH)
        pltpu.sync_copy(srcp_hbm.at[pl.ds(base, CH)], sidx)
        pltpu.sync_copy(dstp_hbm.at[pl.ds(base, CH)], didx)
        for c in range(CH // 16):
            sl = pl.ds(c * 16, 16)
            sidx[sl] = jnp.clip(sidx[sl], 0, NPAD - 1)
            didx[sl] = jnp.clip(didx[sl], 0, NPAD - 1)
        ca = pltpu.async_copy(A_hbm.at[sidx], rowsA, semA)
        cb = pltpu.async_copy(B_hbm.at[didx], rowsB, semB)
        ca.wait()
        cb.wait()

        def radd(r, _):
            for c in range(8):
                sl = pl.ds(c * 16, 16)
                rowsA[r, sl] = rowsA[r, sl] + rowsB[r, sl]
            return 0

        lax.fori_loop(0, CH, radd, 0)
        pltpu.sync_copy(rowsA, G_hbm.at[pl.ds(base, CH)])
        return 0

    lax.fori_loop(0, n_my, ch_body, 0)


def _segmax_body(e_hbm, dstp_hbm, bb2_hbm, aggr_hbm,
                 bbv, dvals, rows, slab, sem):
    NPAD = aggr_hbm.shape[0]
    RW = NPAD // _NW
    CH = 128
    wid = lax.axis_index("s") * _NC + lax.axis_index("c")
    lo = wid * RW

    pltpu.sync_copy(bb2_hbm, bbv)
    v16 = bbv[wid, pl.ds(0, 16)]
    start = v16[0]
    end = v16[1]

    neg = jnp.full((16,), NEG, jnp.float32)

    def init_slab(r, _):
        for c in range(8):
            slab[r, pl.ds(c * 16, 16)] = neg
        return 0

    lax.fori_loop(0, RW, init_slab, 0)

    def ch_body(i, _):
        base = pl.multiple_of(start + i * CH, CH)
        cnt = jnp.minimum(end - base, CH)
        ce = pltpu.async_copy(e_hbm.at[pl.ds(base, CH)], rows, sem)
        pltpu.sync_copy(dstp_hbm.at[pl.ds(base, CH)], dvals)
        ce.wait()
        for g in range(CH // 16):
            dv = dvals[pl.ds(g * 16, 16)]
            for j in range(16):
                k = g * 16 + j

                @pl.when(k < cnt)
                def _():
                    ld = dv[j] - lo
                    for c in range(8):
                        sl = pl.ds(c * 16, 16)
                        slab[ld, sl] = jnp.maximum(slab[ld, sl], rows[k, sl])
        return 0

    lax.fori_loop(0, (end - start + CH - 1) // CH, ch_body, 0)
    lo8 = pl.multiple_of(lo, 8)
    pltpu.sync_copy(slab, aggr_hbm.at[pl.ds(lo8, RW)])


def kernel(x, pos, edge_index, Wh1, bh1, Wh2, bh2, Wf1, bf1, Wf2, bf2,
           Wg1, bg1, Wg2, bg2):
    N, D = x.shape
    E = edge_index.shape[1]
    NPM = _NW * 8
    NP = ((N + NPM - 1) // NPM) * NPM
    EP = E + _NW * 128
    ER = 2560

    xp = jnp.pad(x, ((0, NP - N), (0, 0)))
    pos8 = jnp.pad(pos, ((0, NP - N), (0, 5)))
    Wh2p = jnp.pad(Wh2, ((0, 0), (0, 5)))
    bh2p = jnp.pad(bh2, ((0, 5))).reshape(1, 8)
    W1r = jnp.pad(Wf1[:3], ((0, 5), (0, 0)))
    WfX = Wf1[3:]

    BN = 1024
    gn = NP // BN
    A, B = pl.pallas_call(
        _node_pre_body,
        grid=(gn,),
        in_specs=[
            pl.BlockSpec((BN, D), lambda i: (i, 0)),
            pl.BlockSpec((BN, 8), lambda i: (i, 0)),
            pl.BlockSpec((D, 64), lambda i: (0, 0)),
            pl.BlockSpec((1, 64), lambda i: (0, 0)),
            pl.BlockSpec((64, 8), lambda i: (0, 0)),
            pl.BlockSpec((1, 8), lambda i: (0, 0)),
            pl.BlockSpec((D, 128), lambda i: (0, 0)),
            pl.BlockSpec((8, 128), lambda i: (0, 0)),
            pl.BlockSpec((1, 128), lambda i: (0, 0)),
        ],
        out_specs=[
            pl.BlockSpec((BN, 128), lambda i: (i, 0)),
            pl.BlockSpec((BN, 128), lambda i: (i, 0)),
        ],
        out_shape=[
            jax.ShapeDtypeStruct((NP, 128), jnp.float32),
            jax.ShapeDtypeStruct((NP, 128), jnp.float32),
        ],
    )(xp, pos8, Wh1, bh1.reshape(1, 64), Wh2p, bh2p, WfX, W1r,
      bf1.reshape(1, 128))

    src = edge_index[0]
    dst = edge_index[1]
    dstm = jnp.pad(dst, (0, ER * 128 - E), constant_values=NP).reshape(ER, 128)

    RB = 512
    gk = ER // RB
    colcount = pl.pallas_call(
        _k1_body,
        grid=(gk,),
        in_specs=[pl.BlockSpec((RB, 128), lambda i: (i, 0))],
        out_specs=pl.BlockSpec((_NW, 128), lambda i: (0, 0)),
        out_shape=jax.ShapeDtypeStruct((_NW, 128), jnp.float32),
    )(dstm)

    U128 = jnp.triu(jnp.ones((128, 128), jnp.float32), k=1)
    L32 = jnp.tril(jnp.ones((_NW, _NW), jnp.float32), k=-1)
    lanebase, bb2 = pl.pallas_call(
        _k2_body,
        out_shape=[
            jax.ShapeDtypeStruct((_NW, 128), jnp.float32),
            jax.ShapeDtypeStruct((_NW, 16), jnp.int32),
        ],
    )(colcount, U128, L32)

    LRB = jnp.tril(jnp.ones((RB, RB), jnp.float32), k=-1)
    slot = pl.pallas_call(
        _k3_body,
        grid=(gk,),
        in_specs=[
            pl.BlockSpec((RB, 128), lambda i: (i, 0)),
            pl.BlockSpec((_NW, 128), lambda i: (0, 0)),
            pl.BlockSpec((RB, RB), lambda i: (0, 0)),
        ],
        out_specs=pl.BlockSpec((RB, 128), lambda i: (i, 0)),
        out_shape=jax.ShapeDtypeStruct((ER, 128), jnp.int32),
        scratch_shapes=[pltpu.VMEM((_NW, 128), jnp.float32)],
    )(dstm, lanebase, LRB)
    slot_flat = slot.reshape(ER * 128)[:E]

    mesh = plsc.VectorSubcoreMesh(core_axis_name="c", subcore_axis_name="s")

    CHS = 128
    idp, srcp, dstp = pl.kernel(
        _scatter_body,
        out_type=[
            jax.ShapeDtypeStruct((EP,), jnp.int32),
            jax.ShapeDtypeStruct((EP,), jnp.int32),
            jax.ShapeDtypeStruct((EP,), jnp.int32),
        ],
        mesh=mesh,
        scratch_types=[
            pltpu.VMEM((CHS,), jnp.int32),
            pltpu.VMEM((CHS,), jnp.int32),
            pltpu.VMEM((CHS,), jnp.int32),
            pltpu.VMEM((CHS,), jnp.int32),
            pltpu.VMEM((CHS,), jnp.int32),
        ],
    )(src, dst, slot_flat)

    G = pl.kernel(
        _gather_add_body,
        out_type=jax.ShapeDtypeStruct((EP, 128), jnp.float32),
        mesh=mesh,
        scratch_types=[
            pltpu.VMEM((128,), jnp.int32),
            pltpu.VMEM((128,), jnp.int32),
            pltpu.VMEM((128, 128), jnp.float32),
            pltpu.VMEM((128, 128), jnp.float32),
            pltpu.SemaphoreType.DMA,
            pltpu.SemaphoreType.DMA,
        ],
    )(srcp, dstp, A, B)

    BE = 512
    e = pl.pallas_call(
        _edge_mlp_body,
        grid=(EP // BE,),
        in_specs=[
            pl.BlockSpec((BE, 128), lambda i: (i, 0)),
            pl.BlockSpec((128, 128), lambda i: (0, 0)),
            pl.BlockSpec((1, 128), lambda i: (0, 0)),
        ],
        out_specs=pl.BlockSpec((BE, 128), lambda i: (i, 0)),
        out_shape=jax.ShapeDtypeStruct((EP, 128), jnp.float32),
    )(G, Wf2, bf2.reshape(1, 128))

    RW = NP // _NW
    aggr = pl.kernel(
        _segmax_body,
        out_type=jax.ShapeDtypeStruct((NP, 128), jnp.float32),
        mesh=mesh,
        scratch_types=[
            pltpu.VMEM((_NW, 16), jnp.int32),
            pltpu.VMEM((128,), jnp.int32),
            pltpu.VMEM((128, 128), jnp.float32),
            pltpu.VMEM((RW, 128), jnp.float32),
            pltpu.SemaphoreType.DMA,
        ],
    )(e, dstp, bb2)

    out = pl.pallas_call(
        _out_mlp_body,
        grid=(gn,),
        in_specs=[
            pl.BlockSpec((BN, 128), lambda i: (i, 0)),
            pl.BlockSpec((BN, D), lambda i: (i, 0)),
            pl.BlockSpec((128, 128), lambda i: (0, 0)),
            pl.BlockSpec((1, 128), lambda i: (0, 0)),
            pl.BlockSpec((128, D), lambda i: (0, 0)),
            pl.BlockSpec((1, D), lambda i: (0, 0)),
        ],
        out_specs=pl.BlockSpec((BN, D), lambda i: (i, 0)),
        out_shape=jax.ShapeDtypeStruct((NP, D), jnp.float32),
    )(aggr, xp, Wg1, bg1.reshape(1, 128), Wg2, bg2.reshape(1, D))

    return out[:N]


# speedup vs baseline: 2.1236x; 1.3603x over previous
import jax
import jax.numpy as jnp
from jax import lax
from jax.experimental import pallas as pl
from jax.experimental.pallas import tpu as pltpu
from jax.experimental.pallas import tpu_sc as plsc

NEG = -jnp.inf

_SC_INFO = plsc.get_sparse_core_info()
_NC, _NS = _SC_INFO.num_cores, _SC_INFO.num_subcores
_NW = _NC * _NS

_DIV_MUL = 52429


def _node_pre_body(x_ref, pos8_ref, Wh1_ref, bh1_ref, Wh2_ref, bh2_ref,
                   WfX_ref, W1r_ref, bf1_ref, A_ref, B_ref):
    x = x_ref[...]
    h = jnp.maximum(jnp.dot(x, Wh1_ref[...],
                            preferred_element_type=jnp.float32) + bh1_ref[...], 0.0)
    delta8 = jnp.dot(h, Wh2_ref[...], preferred_element_type=jnp.float32) + bh2_ref[...]
    q8 = delta8 - pos8_ref[...]
    B_ref[...] = jnp.dot(q8, W1r_ref[...], preferred_element_type=jnp.float32)
    A_ref[...] = (jnp.dot(x, WfX_ref[...], preferred_element_type=jnp.float32)
                  + jnp.dot(pos8_ref[...], W1r_ref[...], preferred_element_type=jnp.float32)
                  + bf1_ref[...])


def _k1_body(dstm_ref, cc_ref):
    i = pl.program_id(0)

    @pl.when(i == 0)
    def _():
        cc_ref[...] = jnp.zeros_like(cc_ref)

    b = lax.shift_right_logical(dstm_ref[...] * _DIV_MUL, 24)
    for B in range(_NW):
        eq = (b == B).astype(jnp.float32)
        cc_ref[pl.ds(B, 1), :] += jnp.sum(eq, axis=0, keepdims=True)


def _k2_body(cc_ref, U128_ref, L32_ref, lanebase_ref, bb2_ref):
    cc = cc_ref[...]
    laneprefix = jnp.dot(cc, U128_ref[...], preferred_element_type=jnp.float32)
    totals = jnp.sum(cc, axis=1, keepdims=True)
    ru = jnp.floor((totals + 127.0) * (1.0 / 128.0)) * 128.0
    ab = jnp.dot(L32_ref[...], ru, preferred_element_type=jnp.float32)
    lanebase_ref[...] = ab + laneprefix
    ends = ab + totals
    bb2_ref[...] = jnp.concatenate(
        [ab, ends, jnp.zeros((_NW, 14), jnp.float32)], axis=1).astype(jnp.int32)


def _k3_body(dstm_ref, lanebase_ref, L_ref, slot_ref, carry_ref):
    i = pl.program_id(0)

    @pl.when(i == 0)
    def _():
        carry_ref[...] = jnp.zeros_like(carry_ref)

    b = lax.shift_right_logical(dstm_ref[...] * _DIV_MUL, 24)
    acc = jnp.zeros(slot_ref.shape, jnp.float32)
    L = L_ref[...]
    for B in range(_NW):
        eq = (b == B).astype(jnp.float32)
        p = jnp.dot(L, eq, preferred_element_type=jnp.float32)
        base = carry_ref[pl.ds(B, 1), :] + lanebase_ref[pl.ds(B, 1), :]
        acc = acc + eq * (p + base)
        carry_ref[pl.ds(B, 1), :] += jnp.sum(eq, axis=0, keepdims=True)
    slot_ref[...] = acc.astype(jnp.int32)


def _edge_mlp_body(G_ref, Wf2_ref, bf2_ref, e_ref):
    h = jnp.maximum(G_ref[...], 0.0)
    e_ref[...] = jnp.dot(h, Wf2_ref[...], preferred_element_type=jnp.float32) + bf2_ref[...]


def _out_mlp_body(aggr_ref, x_ref, Wg1_ref, bg1_ref, Wg2_ref, bg2_ref, o_ref):
    a = aggr_ref[...]
    a = jnp.where(a == NEG, 0.0, a)
    h = jnp.maximum(jnp.dot(a, Wg1_ref[...],
                            preferred_element_type=jnp.float32) + bg1_ref[...], 0.0)
    o_ref[...] = (jnp.dot(h, Wg2_ref[...], preferred_element_type=jnp.float32)
                  + bg2_ref[...] + x_ref[...])


def _scatter_body(src_hbm, dst_hbm, slot_hbm, srcp_hbm, dstp_hbm,
                  srcb, dstb, slotb2, sem_in, sem_sc):
    E = src_hbm.shape[0]
    BT = 1280
    SUB = BT // 128
    nbt = E // BT
    wid = lax.axis_index("s") * _NC + lax.axis_index("c")
    n_my = (nbt - wid + _NW - 1) // _NW

    def bt_body(i, _):
        base = pl.multiple_of((wid + i * _NW) * BT, 128)
        cs = pltpu.async_copy(src_hbm.at[pl.ds(base, BT)], srcb, sem_in)
        cd = pltpu.async_copy(dst_hbm.at[pl.ds(base, BT)], dstb, sem_in)
        slcs = [pltpu.async_copy(
                    slot_hbm.at[pl.ds(base + s * 128, 128)], slotb2.at[s], sem_in)
                for s in range(SUB)]
        cs.wait()
        cd.wait()
        for c in slcs:
            c.wait()
        outs = []
        for s in range(SUB):
            outs.append(pltpu.async_copy(
                srcb.at[pl.ds(s * 128, 128)], srcp_hbm.at[slotb2.at[s]], sem_sc))
            outs.append(pltpu.async_copy(
                dstb.at[pl.ds(s * 128, 128)], dstp_hbm.at[slotb2.at[s]], sem_sc))
        for c in outs:
            c.wait()
        return 0

    lax.fori_loop(0, n_my, bt_body, 0)


def _gather_add_body(srcp_hbm, dstp_hbm, A_hbm, B_hbm, G_hbm,
                     sidxb, didxb, rA0, rA1, rA2, rB0, rB1, rB2,
                     sem_in, semA, semB, sem_out):
    EP = srcp_hbm.shape[0]
    NPAD = A_hbm.shape[0]
    RING = 3
    BT = RING * 128
    nbt = EP // BT
    wid = lax.axis_index("s") * _NC + lax.axis_index("c")
    n_my = (nbt - wid + _NW - 1) // _NW
    rA = [rA0, rA1, rA2]
    rB = [rB0, rB1, rB2]

    def bt_body(i, _):
        base = pl.multiple_of((wid + i * _NW) * BT, 128)
        ci = pltpu.async_copy(srcp_hbm.at[pl.ds(base, BT)], sidxb, sem_in)
        cj = pltpu.async_copy(dstp_hbm.at[pl.ds(base, BT)], didxb, sem_in)
        ci.wait()
        cj.wait()
        for c in range(BT // 16):
            sl = pl.ds(c * 16, 16)
            sidxb[sl] = jnp.clip(sidxb[sl], 0, NPAD - 1)
            didxb[sl] = jnp.clip(didxb[sl], 0, NPAD - 1)
        gs = []
        for r in range(RING):
            gs.append(pltpu.async_copy(
                A_hbm.at[sidxb.at[pl.ds(r * 128, 128)]], rA[r], semA))
            gs.append(pltpu.async_copy(
                B_hbm.at[didxb.at[pl.ds(r * 128, 128)]], rB[r], semB))
        outs = []
        for r in range(RING):
            gs[2 * r].wait()
            gs[2 * r + 1].wait()

            def radd(k, _, _r=r):
                for c in range(8):
                    sl = pl.ds(c * 16, 16)
                    rA[_r][k, sl] = rA[_r][k, sl] + rB[_r][k, sl]
                return 0

            lax.fori_loop(0, 128, radd, 0)
            outs.append(pltpu.async_copy(
                rA[r], G_hbm.at[pl.ds(base + r * 128, 128)], sem_out))
        for c in outs:
            c.wait()
        return 0

    lax.fori_loop(0, n_my, bt_body, 0)


def _segmax_body(e_hbm, dstp_hbm, bb2_hbm, aggr_hbm,
                 bbv, dv0, dv1, dv2, r0, r1, r2, slab, sem, semd):
    NPAD = aggr_hbm.shape[0]
    RW = NPAD // _NW
    CH = 128
    RING = 3
    wid = lax.axis_index("s") * _NC + lax.axis_index("c")
    lo = wid * RW

    pltpu.sync_copy(bb2_hbm, bbv)
    v16 = bbv[wid, pl.ds(0, 16)]
    start = v16[0]
    end = v16[1]

    neg = jnp.full((16,), NEG, jnp.float32)

    def init_slab(r, _):
        for c in range(8):
            slab[r, pl.ds(c * 16, 16)] = neg
        return 0

    lax.fori_loop(0, RW, init_slab, 0)

    rows = [r0, r1, r2]
    dvs = [dv0, dv1, dv2]
    nch = (end - start + CH - 1) // CH

    def tri_body(i, _):
        copies = []
        for r in range(RING):
            chunk = i * RING + r
            base = pl.multiple_of(start + chunk * CH, CH)

            @pl.when(chunk < nch)
            def _(r=r, base=base):
                copies.append((
                    pltpu.async_copy(e_hbm.at[pl.ds(base, CH)], rows[r], sem),
                    pltpu.async_copy(dstp_hbm.at[pl.ds(base, CH)], dvs[r], semd)))
        for r in range(RING):
            chunk = i * RING + r
            base = pl.multiple_of(start + chunk * CH, CH)
            cnt = end - base

            @pl.when(chunk < nch)
            def _(r=r, base=base, cnt=cnt):
                pltpu.make_async_copy(e_hbm.at[pl.ds(base, CH)], rows[r], sem).wait()
                pltpu.make_async_copy(dstp_hbm.at[pl.ds(base, CH)], dvs[r], semd).wait()

                @pl.when(cnt >= CH)
                def _():
                    def apply16(g, _):
                        dv = dvs[r][pl.ds(g * 16, 16)]
                        for j in range(16):
                            k = g * 16 + j
                            ld = dv[j] - lo
                            for c in range(8):
                                sl = pl.ds(c * 16, 16)
                                slab[ld, sl] = jnp.maximum(slab[ld, sl],
                                                           rows[r][k, sl])
                        return 0

                    lax.fori_loop(0, CH // 16, apply16, 0)

                @pl.when(cnt < CH)
                def _():
                    def apply16t(g, _):
                        dv = dvs[r][pl.ds(g * 16, 16)]
                        for j in range(16):
                            k = g * 16 + j

                            @pl.when(k < cnt)
                            def _(j=j, k=k):
                                ld = dv[j] - lo
                                for c in range(8):
                                    sl = pl.ds(c * 16, 16)
                                    slab[ld, sl] = jnp.maximum(slab[ld, sl],
                                                               rows[r][k, sl])
                        return 0

                    lax.fori_loop(0, (cnt + 15) // 16, apply16t, 0)
        return 0

    lax.fori_loop(0, (nch + RING - 1) // RING, tri_body, 0)
    lo8 = pl.multiple_of(lo, 8)
    pltpu.sync_copy(slab, aggr_hbm.at[pl.ds(lo8, RW)])


def kernel(x, pos, edge_index, Wh1, bh1, Wh2, bh2, Wf1, bf1, Wf2, bf2,
           Wg1, bg1, Wg2, bg2):
    N, D = x.shape
    E = edge_index.shape[1]
    NPM = _NW * 8
    NP = ((N + NPM - 1) // NPM) * NPM
    EP = E + _NW * 128
    ER = 2560

    xp = jnp.pad(x, ((0, NP - N), (0, 0)))
    pos8 = jnp.pad(pos, ((0, NP - N), (0, 5)))
    Wh2p = jnp.pad(Wh2, ((0, 0), (0, 5)))
    bh2p = jnp.pad(bh2, ((0, 5))).reshape(1, 8)
    W1r = jnp.pad(Wf1[:3], ((0, 5), (0, 0)))
    WfX = Wf1[3:]

    BN = 1024
    gn = NP // BN
    A, B = pl.pallas_call(
        _node_pre_body,
        grid=(gn,),
        in_specs=[
            pl.BlockSpec((BN, D), lambda i: (i, 0)),
            pl.BlockSpec((BN, 8), lambda i: (i, 0)),
            pl.BlockSpec((D, 64), lambda i: (0, 0)),
            pl.BlockSpec((1, 64), lambda i: (0, 0)),
            pl.BlockSpec((64, 8), lambda i: (0, 0)),
            pl.BlockSpec((1, 8), lambda i: (0, 0)),
            pl.BlockSpec((D, 128), lambda i: (0, 0)),
            pl.BlockSpec((8, 128), lambda i: (0, 0)),
            pl.BlockSpec((1, 128), lambda i: (0, 0)),
        ],
        out_specs=[
            pl.BlockSpec((BN, 128), lambda i: (i, 0)),
            pl.BlockSpec((BN, 128), lambda i: (i, 0)),
        ],
        out_shape=[
            jax.ShapeDtypeStruct((NP, 128), jnp.float32),
            jax.ShapeDtypeStruct((NP, 128), jnp.float32),
        ],
    )(xp, pos8, Wh1, bh1.reshape(1, 64), Wh2p, bh2p, WfX, W1r,
      bf1.reshape(1, 128))

    src = edge_index[0]
    dst = edge_index[1]
    dstm = jnp.pad(dst, (0, ER * 128 - E), constant_values=NP).reshape(ER, 128)

    RB = 512
    gk = ER // RB
    colcount = pl.pallas_call(
        _k1_body,
        grid=(gk,),
        in_specs=[pl.BlockSpec((RB, 128), lambda i: (i, 0))],
        out_specs=pl.BlockSpec((_NW, 128), lambda i: (0, 0)),
        out_shape=jax.ShapeDtypeStruct((_NW, 128), jnp.float32),
    )(dstm)

    U128 = jnp.triu(jnp.ones((128, 128), jnp.float32), k=1)
    L32 = jnp.tril(jnp.ones((_NW, _NW), jnp.float32), k=-1)
    lanebase, bb2 = pl.pallas_call(
        _k2_body,
        out_shape=[
            jax.ShapeDtypeStruct((_NW, 128), jnp.float32),
            jax.ShapeDtypeStruct((_NW, 16), jnp.int32),
        ],
    )(colcount, U128, L32)

    LRB = jnp.tril(jnp.ones((RB, RB), jnp.float32), k=-1)
    slot = pl.pallas_call(
        _k3_body,
        grid=(gk,),
        in_specs=[
            pl.BlockSpec((RB, 128), lambda i: (i, 0)),
            pl.BlockSpec((_NW, 128), lambda i: (0, 0)),
            pl.BlockSpec((RB, RB), lambda i: (0, 0)),
        ],
        out_specs=pl.BlockSpec((RB, 128), lambda i: (i, 0)),
        out_shape=jax.ShapeDtypeStruct((ER, 128), jnp.int32),
        scratch_shapes=[pltpu.VMEM((_NW, 128), jnp.float32)],
    )(dstm, lanebase, LRB)
    slot_flat = slot.reshape(ER * 128)[:E]

    mesh = plsc.VectorSubcoreMesh(core_axis_name="c", subcore_axis_name="s")

    srcp, dstp = pl.kernel(
        _scatter_body,
        out_type=[
            jax.ShapeDtypeStruct((EP,), jnp.int32),
            jax.ShapeDtypeStruct((EP,), jnp.int32),
        ],
        mesh=mesh,
        scratch_types=[
            pltpu.VMEM((1280,), jnp.int32),
            pltpu.VMEM((1280,), jnp.int32),
            pltpu.VMEM((10, 128), jnp.int32),
            pltpu.SemaphoreType.DMA,
            pltpu.SemaphoreType.DMA,
        ],
    )(src, dst, slot_flat)

    G = pl.kernel(
        _gather_add_body,
        out_type=jax.ShapeDtypeStruct((EP, 128), jnp.float32),
        mesh=mesh,
        scratch_types=[
            pltpu.VMEM((384,), jnp.int32),
            pltpu.VMEM((384,), jnp.int32),
            pltpu.VMEM((128, 128), jnp.float32),
            pltpu.VMEM((128, 128), jnp.float32),
            pltpu.VMEM((128, 128), jnp.float32),
            pltpu.VMEM((128, 128), jnp.float32),
            pltpu.VMEM((128, 128), jnp.float32),
            pltpu.VMEM((128, 128), jnp.float32),
            pltpu.SemaphoreType.DMA,
            pltpu.SemaphoreType.DMA,
            pltpu.SemaphoreType.DMA,
            pltpu.SemaphoreType.DMA,
        ],
    )(srcp, dstp, A, B)

    BE = 512
    e = pl.pallas_call(
        _edge_mlp_body,
        grid=(EP // BE,),
        in_specs=[
            pl.BlockSpec((BE, 128), lambda i: (i, 0)),
            pl.BlockSpec((128, 128), lambda i: (0, 0)),
            pl.BlockSpec((1, 128), lambda i: (0, 0)),
        ],
        out_specs=pl.BlockSpec((BE, 128), lambda i: (i, 0)),
        out_shape=jax.ShapeDtypeStruct((EP, 128), jnp.float32),
    )(G, Wf2, bf2.reshape(1, 128))

    RW = NP // _NW
    aggr = pl.kernel(
        _segmax_body,
        out_type=jax.ShapeDtypeStruct((NP, 128), jnp.float32),
        mesh=mesh,
        scratch_types=[
            pltpu.VMEM((_NW, 16), jnp.int32),
            pltpu.VMEM((128,), jnp.int32),
            pltpu.VMEM((128,), jnp.int32),
            pltpu.VMEM((128,), jnp.int32),
            pltpu.VMEM((128, 128), jnp.float32),
            pltpu.VMEM((128, 128), jnp.float32),
            pltpu.VMEM((128, 128), jnp.float32),
            pltpu.VMEM((RW, 128), jnp.float32),
            pltpu.SemaphoreType.DMA,
            pltpu.SemaphoreType.DMA,
        ],
    )(e, dstp, bb2)

    out = pl.pallas_call(
        _out_mlp_body,
        grid=(gn,),
        in_specs=[
            pl.BlockSpec((BN, 128), lambda i: (i, 0)),
            pl.BlockSpec((BN, D), lambda i: (i, 0)),
            pl.BlockSpec((128, 128), lambda i: (0, 0)),
            pl.BlockSpec((1, 128), lambda i: (0, 0)),
            pl.BlockSpec((128, D), lambda i: (0, 0)),
            pl.BlockSpec((1, D), lambda i: (0, 0)),
        ],
        out_specs=pl.BlockSpec((BN, D), lambda i: (i, 0)),
        out_shape=jax.ShapeDtypeStruct((NP, D), jnp.float32),
    )(aggr, xp, Wg1, bg1.reshape(1, 128), Wg2, bg2.reshape(1, D))

    return out[:N]


# speedup vs baseline: 2.4618x; 1.1592x over previous
import jax
import jax.numpy as jnp
from jax import lax
from jax.experimental import pallas as pl
from jax.experimental.pallas import tpu as pltpu
from jax.experimental.pallas import tpu_sc as plsc

NEG = -jnp.inf

_SC_INFO = plsc.get_sparse_core_info()
_NC, _NS = _SC_INFO.num_cores, _SC_INFO.num_subcores
_NW = _NC * _NS

_DIV_MUL = 52429


def _node_pre_body(x_ref, pos8_ref, Wh1_ref, bh1_ref, Wh2_ref, bh2_ref,
                   WfX_ref, W1r_ref, bf1_ref, A_ref, B_ref):
    x = x_ref[...]
    h = jnp.maximum(jnp.dot(x, Wh1_ref[...],
                            preferred_element_type=jnp.float32) + bh1_ref[...], 0.0)
    delta8 = jnp.dot(h, Wh2_ref[...], preferred_element_type=jnp.float32) + bh2_ref[...]
    q8 = delta8 - pos8_ref[...]
    B_ref[...] = jnp.dot(q8, W1r_ref[...], preferred_element_type=jnp.float32)
    A_ref[...] = (jnp.dot(x, WfX_ref[...], preferred_element_type=jnp.float32)
                  + jnp.dot(pos8_ref[...], W1r_ref[...], preferred_element_type=jnp.float32)
                  + bf1_ref[...])


def _k1_body(dstm_ref, cc_ref):
    i = pl.program_id(0)

    @pl.when(i == 0)
    def _():
        cc_ref[...] = jnp.zeros_like(cc_ref)

    b = lax.shift_right_logical(dstm_ref[...] * _DIV_MUL, 24)
    for B in range(_NW):
        eq = (b == B).astype(jnp.float32)
        cc_ref[pl.ds(B, 1), :] += jnp.sum(eq, axis=0, keepdims=True)


def _k2_body(cc_ref, U128_ref, L32_ref, lanebase_ref, bb2_ref):
    cc = cc_ref[...]
    laneprefix = jnp.dot(cc, U128_ref[...], preferred_element_type=jnp.float32)
    totals = jnp.sum(cc, axis=1, keepdims=True)
    ru = jnp.floor((totals + 127.0) * (1.0 / 128.0)) * 128.0
    ab = jnp.dot(L32_ref[...], ru, preferred_element_type=jnp.float32)
    lanebase_ref[...] = ab + laneprefix
    ends = ab + totals
    bb2_ref[...] = jnp.concatenate(
        [ab, ends, jnp.zeros((_NW, 14), jnp.float32)], axis=1).astype(jnp.int32)


def _k3_body(dstm_ref, lanebase_ref, L_ref, slot_ref, carry_ref):
    i = pl.program_id(0)

    @pl.when(i == 0)
    def _():
        carry_ref[...] = jnp.zeros_like(carry_ref)

    b = lax.shift_right_logical(dstm_ref[...] * _DIV_MUL, 24)
    acc = jnp.zeros(slot_ref.shape, jnp.float32)
    L = L_ref[...]
    for B in range(_NW):
        eq = (b == B).astype(jnp.float32)
        p = jnp.dot(L, eq, preferred_element_type=jnp.float32)
        base = carry_ref[pl.ds(B, 1), :] + lanebase_ref[pl.ds(B, 1), :]
        acc = acc + eq * (p + base)
        carry_ref[pl.ds(B, 1), :] += jnp.sum(eq, axis=0, keepdims=True)
    slot_ref[...] = acc.astype(jnp.int32)


def _edge_mlp_body(G_ref, Wf2_ref, bf2_ref, e_ref):
    h = jnp.maximum(G_ref[...], 0.0)
    e_ref[...] = jnp.dot(h, Wf2_ref[...], preferred_element_type=jnp.float32) + bf2_ref[...]


def _out_mlp_body(aggr_ref, x_ref, Wg1_ref, bg1_ref, Wg2_ref, bg2_ref, o_ref):
    a = aggr_ref[...]
    a = jnp.where(a == NEG, 0.0, a)
    h = jnp.maximum(jnp.dot(a, Wg1_ref[...],
                            preferred_element_type=jnp.float32) + bg1_ref[...], 0.0)
    o_ref[...] = (jnp.dot(h, Wg2_ref[...], preferred_element_type=jnp.float32)
                  + bg2_ref[...] + x_ref[...])


def _scatter_body(src_hbm, dst_hbm, slot_hbm, vp_hbm,
                  srcb, dstb, slotb, vb2, slotb2, sem_in, sem_sc):
    E = src_hbm.shape[0]
    BT = 2560
    SUB = BT // 128
    nbt = E // BT
    wid = lax.axis_index("s") * _NC + lax.axis_index("c")
    n_my = (nbt - wid + _NW - 1) // _NW

    def bt_body(i, _):
        base = pl.multiple_of((wid + i * _NW) * BT, 128)
        cs = pltpu.async_copy(src_hbm.at[pl.ds(base, BT)], srcb, sem_in)
        cd = pltpu.async_copy(dst_hbm.at[pl.ds(base, BT)], dstb, sem_in)
        cl = pltpu.async_copy(slot_hbm.at[pl.ds(base, BT)], slotb, sem_in)
        cs.wait()
        cd.wait()
        cl.wait()
        for r in range(SUB):
            for c in range(8):
                src1 = pl.ds(r * 128 + c * 16, 16)
                dst1 = pl.ds(c * 16, 16)
                vb2[r, dst1] = srcb[src1] * 16384 + dstb[src1]
                slotb2[r, dst1] = slotb[src1]
        outs = [pltpu.async_copy(vb2.at[r], vp_hbm.at[slotb2.at[r]], sem_sc)
                for r in range(SUB)]
        for c in outs:
            c.wait()
        return 0

    lax.fori_loop(0, n_my, bt_body, 0)


def _gather_add_body(vp_hbm, A_hbm, B_hbm, G_hbm,
                     vb, sidx2, didx2, rA, rB,
                     sem_in, semA, semB, sem_out):
    EPC = G_hbm.shape[0]
    NPAD = A_hbm.shape[0]
    K = 3
    BT = K * 128
    nbt = EPC // K
    wid = lax.axis_index("s") * _NC + lax.axis_index("c")
    n_my = (nbt - wid + _NW - 1) // _NW

    def bt_body(i, _):
        bt = wid + i * _NW
        base = pl.multiple_of(bt * BT, 128)
        pltpu.async_copy(vp_hbm.at[pl.ds(base, BT)], vb, sem_in).wait()
        for r in range(K):
            for c in range(8):
                src1 = pl.ds(r * 128 + c * 16, 16)
                dst1 = pl.ds(c * 16, 16)
                v = vb[src1]
                sidx2[r, dst1] = jnp.clip(lax.shift_right_logical(v, 14),
                                          0, NPAD - 1)
                didx2[r, dst1] = jnp.clip(v & 16383, 0, NPAD - 1)
        gs = []
        for r in range(K):
            gs.append(pltpu.async_copy(A_hbm.at[sidx2.at[r]], rA.at[r], semA))
            gs.append(pltpu.async_copy(B_hbm.at[didx2.at[r]], rB.at[r], semB))
        for c in gs:
            c.wait()

        def radd(k, _):
            r3 = k >> 7
            kk = k & 127
            for c in range(8):
                sl = pl.ds(c * 16, 16)
                rA[r3, kk, sl] = rA[r3, kk, sl] + rB[r3, kk, sl]
            return 0

        lax.fori_loop(0, BT, radd, 0)
        pltpu.async_copy(rA, G_hbm.at[pl.ds(bt * K, K)], sem_out).wait()
        return 0

    lax.fori_loop(0, n_my, bt_body, 0)


def _segmax_body(e_hbm, vp_hbm, bb2_hbm, aggr_hbm,
                 bbv, dv0, dv1, r0, r1, slab, sem, semd):
    NPAD = aggr_hbm.shape[0]
    RW = NPAD // _NW
    SCH = 256
    wid = lax.axis_index("s") * _NC + lax.axis_index("c")
    lo = wid * RW

    pltpu.sync_copy(bb2_hbm, bbv)
    v16 = bbv[wid, pl.ds(0, 16)]
    start = v16[0]
    end = v16[1]

    neg = jnp.full((16,), NEG, jnp.float32)

    def init_slab(r, _):
        for c in range(8):
            slab[r, pl.ds(c * 16, 16)] = neg
        return 0

    lax.fori_loop(0, RW, init_slab, 0)

    rows = [r0, r1]
    dvs = [dv0, dv1]
    nsch = (end - start + SCH - 1) // SCH

    def pair_body(i, _):
        for p in range(2):
            sc = i * 2 + p
            base = pl.multiple_of(start + sc * SCH, 128)

            @pl.when(sc < nsch)
            def _(p=p, base=base):
                pltpu.async_copy(e_hbm.at[pl.ds(base, SCH)], rows[p], sem)
                pltpu.async_copy(vp_hbm.at[pl.ds(base, SCH)], dvs[p], semd)
        for p in range(2):
            sc = i * 2 + p
            base = pl.multiple_of(start + sc * SCH, 128)
            cnt = end - base

            @pl.when(sc < nsch)
            def _(p=p, base=base, cnt=cnt):
                pltpu.make_async_copy(e_hbm.at[pl.ds(base, SCH)], rows[p], sem).wait()
                pltpu.make_async_copy(vp_hbm.at[pl.ds(base, SCH)], dvs[p], semd).wait()

                @pl.when(cnt >= SCH)
                def _():
                    def apply16(g, _):
                        dv = dvs[p][pl.ds(g * 16, 16)] & 16383
                        for j in range(16):
                            k = g * 16 + j
                            ld = dv[j] - lo
                            for c in range(8):
                                sl = pl.ds(c * 16, 16)
                                slab[ld, sl] = jnp.maximum(slab[ld, sl],
                                                           rows[p][k, sl])
                        return 0

                    lax.fori_loop(0, SCH // 16, apply16, 0)

                @pl.when(cnt < SCH)
                def _():
                    def apply16t(g, _):
                        dv = dvs[p][pl.ds(g * 16, 16)] & 16383
                        for j in range(16):
                            k = g * 16 + j

                            @pl.when(k < cnt)
                            def _(j=j, k=k):
                                ld = dv[j] - lo
                                for c in range(8):
                                    sl = pl.ds(c * 16, 16)
                                    slab[ld, sl] = jnp.maximum(slab[ld, sl],
                                                               rows[p][k, sl])
                        return 0

                    lax.fori_loop(0, (cnt + 15) // 16, apply16t, 0)
        return 0

    lax.fori_loop(0, (nsch + 1) // 2, pair_body, 0)
    lo8 = pl.multiple_of(lo, 8)
    pltpu.sync_copy(slab, aggr_hbm.at[pl.ds(lo8, RW)])


def kernel(x, pos, edge_index, Wh1, bh1, Wh2, bh2, Wf1, bf1, Wf2, bf2,
           Wg1, bg1, Wg2, bg2):
    N, D = x.shape
    E = edge_index.shape[1]
    NPM = _NW * 8
    NP = ((N + NPM - 1) // NPM) * NPM
    EP = E + _NW * 128
    ER = 2560

    xp = jnp.pad(x, ((0, NP - N), (0, 0)))
    pos8 = jnp.pad(pos, ((0, NP - N), (0, 5)))
    Wh2p = jnp.pad(Wh2, ((0, 0), (0, 5)))
    bh2p = jnp.pad(bh2, ((0, 5))).reshape(1, 8)
    W1r = jnp.pad(Wf1[:3], ((0, 5), (0, 0)))
    WfX = Wf1[3:]

    BN = 1024
    gn = NP // BN
    A, B = pl.pallas_call(
        _node_pre_body,
        grid=(gn,),
        in_specs=[
            pl.BlockSpec((BN, D), lambda i: (i, 0)),
            pl.BlockSpec((BN, 8), lambda i: (i, 0)),
            pl.BlockSpec((D, 64), lambda i: (0, 0)),
            pl.BlockSpec((1, 64), lambda i: (0, 0)),
            pl.BlockSpec((64, 8), lambda i: (0, 0)),
            pl.BlockSpec((1, 8), lambda i: (0, 0)),
            pl.BlockSpec((D, 128), lambda i: (0, 0)),
            pl.BlockSpec((8, 128), lambda i: (0, 0)),
            pl.BlockSpec((1, 128), lambda i: (0, 0)),
        ],
        out_specs=[
            pl.BlockSpec((BN, 128), lambda i: (i, 0)),
            pl.BlockSpec((BN, 128), lambda i: (i, 0)),
        ],
        out_shape=[
            jax.ShapeDtypeStruct((NP, 128), jnp.float32),
            jax.ShapeDtypeStruct((NP, 128), jnp.float32),
        ],
    )(xp, pos8, Wh1, bh1.reshape(1, 64), Wh2p, bh2p, WfX, W1r,
      bf1.reshape(1, 128))

    src = edge_index[0]
    dst = edge_index[1]
    dstm = jnp.pad(dst, (0, ER * 128 - E), constant_values=NP).reshape(ER, 128)

    RB = 512
    gk = ER // RB
    colcount = pl.pallas_call(
        _k1_body,
        grid=(gk,),
        in_specs=[pl.BlockSpec((RB, 128), lambda i: (i, 0))],
        out_specs=pl.BlockSpec((_NW, 128), lambda i: (0, 0)),
        out_shape=jax.ShapeDtypeStruct((_NW, 128), jnp.float32),
    )(dstm)

    U128 = jnp.triu(jnp.ones((128, 128), jnp.float32), k=1)
    L32 = jnp.tril(jnp.ones((_NW, _NW), jnp.float32), k=-1)
    lanebase, bb2 = pl.pallas_call(
        _k2_body,
        out_shape=[
            jax.ShapeDtypeStruct((_NW, 128), jnp.float32),
            jax.ShapeDtypeStruct((_NW, 16), jnp.int32),
        ],
    )(colcount, U128, L32)

    LRB = jnp.tril(jnp.ones((RB, RB), jnp.float32), k=-1)
    slot = pl.pallas_call(
        _k3_body,
        grid=(gk,),
        in_specs=[
            pl.BlockSpec((RB, 128), lambda i: (i, 0)),
            pl.BlockSpec((_NW, 128), lambda i: (0, 0)),
            pl.BlockSpec((RB, RB), lambda i: (0, 0)),
        ],
        out_specs=pl.BlockSpec((RB, 128), lambda i: (i, 0)),
        out_shape=jax.ShapeDtypeStruct((ER, 128), jnp.int32),
        scratch_shapes=[pltpu.VMEM((_NW, 128), jnp.float32)],
    )(dstm, lanebase, LRB)
    slot_flat = slot.reshape(ER * 128)[:E]

    mesh = plsc.VectorSubcoreMesh(core_axis_name="c", subcore_axis_name="s")

    vp = pl.kernel(
        _scatter_body,
        out_type=jax.ShapeDtypeStruct((EP,), jnp.int32),
        mesh=mesh,
        scratch_types=[
            pltpu.VMEM((2560,), jnp.int32),
            pltpu.VMEM((2560,), jnp.int32),
            pltpu.VMEM((2560,), jnp.int32),
            pltpu.VMEM((20, 128), jnp.int32),
            pltpu.VMEM((20, 128), jnp.int32),
            pltpu.SemaphoreType.DMA,
            pltpu.SemaphoreType.DMA,
        ],
    )(src, dst, slot_flat)

    G3 = pl.kernel(
        _gather_add_body,
        out_type=jax.ShapeDtypeStruct((EP // 128, 128, 128), jnp.float32),
        mesh=mesh,
        scratch_types=[
            pltpu.VMEM((384,), jnp.int32),
            pltpu.VMEM((3, 128), jnp.int32),
            pltpu.VMEM((3, 128), jnp.int32),
            pltpu.VMEM((3, 128, 128), jnp.float32),
            pltpu.VMEM((3, 128, 128), jnp.float32),
            pltpu.SemaphoreType.DMA,
            pltpu.SemaphoreType.DMA,
            pltpu.SemaphoreType.DMA,
            pltpu.SemaphoreType.DMA,
        ],
    )(vp, A, B)
    G = G3.reshape(EP, 128)

    BE = 512
    e = pl.pallas_call(
        _edge_mlp_body,
        grid=(EP // BE,),
        in_specs=[
            pl.BlockSpec((BE, 128), lambda i: (i, 0)),
            pl.BlockSpec((128, 128), lambda i: (0, 0)),
            pl.BlockSpec((1, 128), lambda i: (0, 0)),
        ],
        out_specs=pl.BlockSpec((BE, 128), lambda i: (i, 0)),
        out_shape=jax.ShapeDtypeStruct((EP, 128), jnp.float32),
    )(G, Wf2, bf2.reshape(1, 128))

    RW = NP // _NW
    aggr = pl.kernel(
        _segmax_body,
        out_type=jax.ShapeDtypeStruct((NP, 128), jnp.float32),
        mesh=mesh,
        scratch_types=[
            pltpu.VMEM((_NW, 16), jnp.int32),
            pltpu.VMEM((256,), jnp.int32),
            pltpu.VMEM((256,), jnp.int32),
            pltpu.VMEM((256, 128), jnp.float32),
            pltpu.VMEM((256, 128), jnp.float32),
            pltpu.VMEM((RW, 128), jnp.float32),
            pltpu.SemaphoreType.DMA,
            pltpu.SemaphoreType.DMA,
        ],
    )(e, vp, bb2)

    out = pl.pallas_call(
        _out_mlp_body,
        grid=(gn,),
        in_specs=[
            pl.BlockSpec((BN, 128), lambda i: (i, 0)),
            pl.BlockSpec((BN, D), lambda i: (i, 0)),
            pl.BlockSpec((128, 128), lambda i: (0, 0)),
            pl.BlockSpec((1, 128), lambda i: (0, 0)),
            pl.BlockSpec((128, D), lambda i: (0, 0)),
            pl.BlockSpec((1, D), lambda i: (0, 0)),
        ],
        out_specs=pl.BlockSpec((BN, D), lambda i: (i, 0)),
        out_shape=jax.ShapeDtypeStruct((NP, D), jnp.float32),
    )(aggr, xp, Wg1, bg1.reshape(1, 128), Wg2, bg2.reshape(1, D))

    return out[:N]


# speedup vs baseline: 2.4799x; 1.0074x over previous
import jax
import jax.numpy as jnp
from jax import lax
from jax.experimental import pallas as pl
from jax.experimental.pallas import tpu as pltpu
from jax.experimental.pallas import tpu_sc as plsc

NEG = -jnp.inf

_SC_INFO = plsc.get_sparse_core_info()
_NC, _NS = _SC_INFO.num_cores, _SC_INFO.num_subcores
_NW = _NC * _NS

_DIV_MUL = 52429


def _node_pre_body(x_ref, pos8_ref, Wh1_ref, bh1_ref, Wh2_ref, bh2_ref,
                   WfX_ref, W1r_ref, bf1_ref, A_ref, B_ref):
    x = x_ref[...]
    h = jnp.maximum(jnp.dot(x, Wh1_ref[...],
                            preferred_element_type=jnp.float32) + bh1_ref[...], 0.0)
    delta8 = jnp.dot(h, Wh2_ref[...], preferred_element_type=jnp.float32) + bh2_ref[...]
    q8 = delta8 - pos8_ref[...]
    B_ref[...] = jnp.dot(q8, W1r_ref[...], preferred_element_type=jnp.float32)
    A_ref[...] = (jnp.dot(x, WfX_ref[...], preferred_element_type=jnp.float32)
                  + jnp.dot(pos8_ref[...], W1r_ref[...], preferred_element_type=jnp.float32)
                  + bf1_ref[...])


def _k1_body(dstm_ref, cc_ref):
    i = pl.program_id(0)

    @pl.when(i == 0)
    def _():
        cc_ref[...] = jnp.zeros_like(cc_ref)

    b = lax.shift_right_logical(dstm_ref[...] * _DIV_MUL, 24)
    for B in range(_NW):
        eq = (b == B).astype(jnp.float32)
        cc_ref[pl.ds(B, 1), :] += jnp.sum(eq, axis=0, keepdims=True)


def _k2_body(cc_ref, U128_ref, L32_ref, lanebase_ref, bb2_ref):
    cc = cc_ref[...]
    laneprefix = jnp.dot(cc, U128_ref[...], preferred_element_type=jnp.float32)
    totals = jnp.sum(cc, axis=1, keepdims=True)
    ru = jnp.floor((totals + 127.0) * (1.0 / 128.0)) * 128.0
    ab = jnp.dot(L32_ref[...], ru, preferred_element_type=jnp.float32)
    lanebase_ref[...] = ab + laneprefix
    ends = ab + totals
    bb2_ref[...] = jnp.concatenate(
        [ab, ends, jnp.zeros((_NW, 14), jnp.float32)], axis=1).astype(jnp.int32)


def _k3_body(dstm_ref, lanebase_ref, L_ref, slot_ref, carry_ref):
    i = pl.program_id(0)

    @pl.when(i == 0)
    def _():
        carry_ref[...] = jnp.zeros_like(carry_ref)

    b = lax.shift_right_logical(dstm_ref[...] * _DIV_MUL, 24)
    acc = jnp.zeros(slot_ref.shape, jnp.float32)
    L = L_ref[...]
    for B in range(_NW):
        eq = (b == B).astype(jnp.float32)
        p = jnp.dot(L, eq, preferred_element_type=jnp.float32)
        base = carry_ref[pl.ds(B, 1), :] + lanebase_ref[pl.ds(B, 1), :]
        acc = acc + eq * (p + base)
        carry_ref[pl.ds(B, 1), :] += jnp.sum(eq, axis=0, keepdims=True)
    slot_ref[...] = acc.astype(jnp.int32)


def _edge_mlp_body(G_ref, Wf2_ref, bf2_ref, e_ref):
    h = jnp.maximum(G_ref[...], 0.0)
    e_ref[...] = jnp.dot(h, Wf2_ref[...], preferred_element_type=jnp.float32) + bf2_ref[...]


def _out_mlp_body(aggr_ref, x_ref, Wg1_ref, bg1_ref, Wg2_ref, bg2_ref, o_ref):
    a = aggr_ref[...]
    a = jnp.where(jnp.isneginf(a), 0.0, a)
    h = jnp.maximum(jnp.dot(a, Wg1_ref[...],
                            preferred_element_type=jnp.float32) + bg1_ref[...], 0.0)
    o_ref[...] = (jnp.dot(h, Wg2_ref[...], preferred_element_type=jnp.float32)
                  + bg2_ref[...] + x_ref[...])


def _scatter_body(src_hbm, dst_hbm, slot_hbm, vp_hbm,
                  srcb, dstb, slotb, vb2, slotb2, sem_in, sem_sc):
    E = src_hbm.shape[0]
    BT = 2560
    SUB = BT // 128
    nbt = E // BT
    wid = lax.axis_index("s") * _NC + lax.axis_index("c")
    n_my = (nbt - wid + _NW - 1) // _NW

    def bt_body(i, _):
        base = pl.multiple_of((wid + i * _NW) * BT, 128)
        cs = pltpu.async_copy(src_hbm.at[pl.ds(base, BT)], srcb, sem_in)
        cd = pltpu.async_copy(dst_hbm.at[pl.ds(base, BT)], dstb, sem_in)
        cl = pltpu.async_copy(slot_hbm.at[pl.ds(base, BT)], slotb, sem_in)
        cs.wait()
        cd.wait()
        cl.wait()
        for r in range(SUB):
            for c in range(8):
                src1 = pl.ds(r * 128 + c * 16, 16)
                dst1 = pl.ds(c * 16, 16)
                vb2[r, dst1] = srcb[src1] * 16384 + dstb[src1]
                slotb2[r, dst1] = slotb[src1]
        outs = [pltpu.async_copy(vb2.at[r], vp_hbm.at[slotb2.at[r]], sem_sc)
                for r in range(SUB)]
        for c in outs:
            c.wait()
        return 0

    lax.fori_loop(0, n_my, bt_body, 0)


def _gather_add_body(vp_hbm, A_hbm, B_hbm, G_hbm,
                     vb, sidx2, didx2, rA, rB,
                     sem_in, semA, semB, sem_out):
    EPC = G_hbm.shape[0]
    NPAD = A_hbm.shape[0]
    K = 3
    BT = K * 128
    nbt = EPC // K
    wid = lax.axis_index("s") * _NC + lax.axis_index("c")
    n_my = (nbt - wid + _NW - 1) // _NW

    def bt_body(i, _):
        bt = wid + i * _NW
        base = pl.multiple_of(bt * BT, 128)
        pltpu.async_copy(vp_hbm.at[pl.ds(base, BT)], vb, sem_in).wait()
        for r in range(K):
            for c in range(8):
                src1 = pl.ds(r * 128 + c * 16, 16)
                dst1 = pl.ds(c * 16, 16)
                v = vb[src1]
                sidx2[r, dst1] = jnp.clip(lax.shift_right_logical(v, 14),
                                          0, NPAD - 1)
                didx2[r, dst1] = jnp.clip(v & 16383, 0, NPAD - 1)
        gs = []
        for r in range(K):
            gs.append(pltpu.async_copy(A_hbm.at[sidx2.at[r]], rA.at[r], semA))
            gs.append(pltpu.async_copy(B_hbm.at[didx2.at[r]], rB.at[r], semB))
        for c in gs:
            c.wait()

        def radd(k, _):
            r3 = k >> 7
            kk = k & 127
            for c in range(8):
                sl = pl.ds(c * 16, 16)
                rA[r3, kk, sl] = rA[r3, kk, sl] + rB[r3, kk, sl]
            return 0

        lax.fori_loop(0, BT, radd, 0)
        pltpu.async_copy(rA, G_hbm.at[pl.ds(bt * K, K)], sem_out).wait()
        return 0

    lax.fori_loop(0, n_my, bt_body, 0)


def _segmax_body(e_hbm, vp_hbm, bb2_hbm, aggr_hbm,
                 bbv, dv0, dv1, r0, r1, slab, sem, semd):
    NPAD = aggr_hbm.shape[0]
    RW = NPAD // _NW
    SCH = 256
    wid = lax.axis_index("s") * _NC + lax.axis_index("c")
    lo = wid * RW

    pltpu.sync_copy(bb2_hbm, bbv)
    v16 = bbv[wid, pl.ds(0, 16)]
    start = v16[0]
    end = v16[1]

    neg = jnp.full((16,), NEG, jnp.float32)

    def init_slab(r, _):
        for c in range(8):
            slab[r, pl.ds(c * 16, 16)] = neg
        return 0

    lax.fori_loop(0, RW, init_slab, 0)

    rows = [r0, r1]
    dvs = [dv0, dv1]
    nsch = (end - start + SCH - 1) // SCH

    def pair_body(i, _):
        for p in range(2):
            sc = i * 2 + p
            base = pl.multiple_of(start + sc * SCH, 128)

            @pl.when(sc < nsch)
            def _(p=p, base=base):
                pltpu.async_copy(e_hbm.at[pl.ds(base, SCH)], rows[p], sem)
                pltpu.async_copy(vp_hbm.at[pl.ds(base, SCH)], dvs[p], semd)
        for p in range(2):
            sc = i * 2 + p
            base = pl.multiple_of(start + sc * SCH, 128)
            cnt = end - base

            @pl.when(sc < nsch)
            def _(p=p, base=base, cnt=cnt):
                pltpu.make_async_copy(e_hbm.at[pl.ds(base, SCH)], rows[p], sem).wait()
                pltpu.make_async_copy(vp_hbm.at[pl.ds(base, SCH)], dvs[p], semd).wait()

                @pl.when(cnt >= SCH)
                def _():
                    def apply16(g, _):
                        dv = dvs[p][pl.ds(g * 16, 16)] & 16383
                        for j in range(16):
                            k = g * 16 + j
                            ld = dv[j] - lo
                            for c in range(8):
                                sl = pl.ds(c * 16, 16)
                                slab[ld, sl] = jnp.maximum(slab[ld, sl],
                                                           rows[p][k, sl])
                        return 0

                    lax.fori_loop(0, SCH // 16, apply16, 0)

                @pl.when(cnt < SCH)
                def _():
                    def apply16t(g, _):
                        dv = dvs[p][pl.ds(g * 16, 16)] & 16383
                        for j in range(16):
                            k = g * 16 + j

                            @pl.when(k < cnt)
                            def _(j=j, k=k):
                                ld = dv[j] - lo
                                for c in range(8):
                                    sl = pl.ds(c * 16, 16)
                                    slab[ld, sl] = jnp.maximum(slab[ld, sl],
                                                               rows[p][k, sl])
                        return 0

                    lax.fori_loop(0, (cnt + 15) // 16, apply16t, 0)
        return 0

    lax.fori_loop(0, (nsch + 1) // 2, pair_body, 0)
    lo8 = pl.multiple_of(lo, 8)
    pltpu.sync_copy(slab, aggr_hbm.at[pl.ds(lo8, RW)])


def kernel(x, pos, edge_index, Wh1, bh1, Wh2, bh2, Wf1, bf1, Wf2, bf2,
           Wg1, bg1, Wg2, bg2):
    N, D = x.shape
    E = edge_index.shape[1]
    NPM = _NW * 8
    NP = ((N + NPM - 1) // NPM) * NPM
    EP = E + _NW * 128
    ER = 2560

    xp = jnp.pad(x, ((0, NP - N), (0, 0)))
    pos8 = jnp.pad(pos, ((0, NP - N), (0, 5)))
    Wh2p = jnp.pad(Wh2, ((0, 0), (0, 5)))
    bh2p = jnp.pad(bh2, ((0, 5))).reshape(1, 8)
    W1r = jnp.pad(Wf1[:3], ((0, 5), (0, 0)))
    WfX = Wf1[3:]

    BN = 1024
    gn = NP // BN
    A, B = pl.pallas_call(
        _node_pre_body,
        grid=(gn,),
        in_specs=[
            pl.BlockSpec((BN, D), lambda i: (i, 0)),
            pl.BlockSpec((BN, 8), lambda i: (i, 0)),
            pl.BlockSpec((D, 64), lambda i: (0, 0)),
            pl.BlockSpec((1, 64), lambda i: (0, 0)),
            pl.BlockSpec((64, 8), lambda i: (0, 0)),
            pl.BlockSpec((1, 8), lambda i: (0, 0)),
            pl.BlockSpec((D, 128), lambda i: (0, 0)),
            pl.BlockSpec((8, 128), lambda i: (0, 0)),
            pl.BlockSpec((1, 128), lambda i: (0, 0)),
        ],
        out_specs=[
            pl.BlockSpec((BN, 128), lambda i: (i, 0)),
            pl.BlockSpec((BN, 128), lambda i: (i, 0)),
        ],
        out_shape=[
            jax.ShapeDtypeStruct((NP, 128), jnp.float32),
            jax.ShapeDtypeStruct((NP, 128), jnp.float32),
        ],
    )(xp, pos8, Wh1, bh1.reshape(1, 64), Wh2p, bh2p, WfX, W1r,
      bf1.reshape(1, 128))

    src = edge_index[0]
    dst = edge_index[1]
    dstm = jnp.pad(dst, (0, ER * 128 - E), constant_values=NP).reshape(ER, 128)

    RB = 128
    gk = ER // RB
    colcount = pl.pallas_call(
        _k1_body,
        grid=(gk,),
        in_specs=[pl.BlockSpec((RB, 128), lambda i: (i, 0))],
        out_specs=pl.BlockSpec((_NW, 128), lambda i: (0, 0)),
        out_shape=jax.ShapeDtypeStruct((_NW, 128), jnp.float32),
    )(dstm)

    U128 = jnp.triu(jnp.ones((128, 128), jnp.float32), k=1)
    L32 = jnp.tril(jnp.ones((_NW, _NW), jnp.float32), k=-1)
    lanebase, bb2 = pl.pallas_call(
        _k2_body,
        out_shape=[
            jax.ShapeDtypeStruct((_NW, 128), jnp.float32),
            jax.ShapeDtypeStruct((_NW, 16), jnp.int32),
        ],
    )(colcount, U128, L32)

    LRB = jnp.tril(jnp.ones((RB, RB), jnp.float32), k=-1)
    slot = pl.pallas_call(
        _k3_body,
        grid=(gk,),
        in_specs=[
            pl.BlockSpec((RB, 128), lambda i: (i, 0)),
            pl.BlockSpec((_NW, 128), lambda i: (0, 0)),
            pl.BlockSpec((RB, RB), lambda i: (0, 0)),
        ],
        out_specs=pl.BlockSpec((RB, 128), lambda i: (i, 0)),
        out_shape=jax.ShapeDtypeStruct((ER, 128), jnp.int32),
        scratch_shapes=[pltpu.VMEM((_NW, 128), jnp.float32)],
    )(dstm, lanebase, LRB)
    slot_flat = slot.reshape(ER * 128)[:E]

    mesh = plsc.VectorSubcoreMesh(core_axis_name="c", subcore_axis_name="s")

    vp = pl.kernel(
        _scatter_body,
        out_type=jax.ShapeDtypeStruct((EP,), jnp.int32),
        mesh=mesh,
        scratch_types=[
            pltpu.VMEM((2560,), jnp.int32),
            pltpu.VMEM((2560,), jnp.int32),
            pltpu.VMEM((2560,), jnp.int32),
            pltpu.VMEM((20, 128), jnp.int32),
            pltpu.VMEM((20, 128), jnp.int32),
            pltpu.SemaphoreType.DMA,
            pltpu.SemaphoreType.DMA,
        ],
    )(src, dst, slot_flat)

    G3 = pl.kernel(
        _gather_add_body,
        out_type=jax.ShapeDtypeStruct((EP // 128, 128, 128), jnp.float32),
        mesh=mesh,
        scratch_types=[
            pltpu.VMEM((384,), jnp.int32),
            pltpu.VMEM((3, 128), jnp.int32),
            pltpu.VMEM((3, 128), jnp.int32),
            pltpu.VMEM((3, 128, 128), jnp.float32),
            pltpu.VMEM((3, 128, 128), jnp.float32),
            pltpu.SemaphoreType.DMA,
            pltpu.SemaphoreType.DMA,
            pltpu.SemaphoreType.DMA,
            pltpu.SemaphoreType.DMA,
        ],
    )(vp, A, B)
    G = G3.reshape(EP, 128)

    BE = 512
    e = pl.pallas_call(
        _edge_mlp_body,
        grid=(EP // BE,),
        in_specs=[
            pl.BlockSpec((BE, 128), lambda i: (i, 0)),
            pl.BlockSpec((128, 128), lambda i: (0, 0)),
            pl.BlockSpec((1, 128), lambda i: (0, 0)),
        ],
        out_specs=pl.BlockSpec((BE, 128), lambda i: (i, 0)),
        out_shape=jax.ShapeDtypeStruct((EP, 128), jnp.float32),
    )(G, Wf2, bf2.reshape(1, 128))

    RW = NP // _NW
    aggr = pl.kernel(
        _segmax_body,
        out_type=jax.ShapeDtypeStruct((NP, 128), jnp.float32),
        mesh=mesh,
        scratch_types=[
            pltpu.VMEM((_NW, 16), jnp.int32),
            pltpu.VMEM((256,), jnp.int32),
            pltpu.VMEM((256,), jnp.int32),
            pltpu.VMEM((256, 128), jnp.float32),
            pltpu.VMEM((256, 128), jnp.float32),
            pltpu.VMEM((RW, 128), jnp.float32),
            pltpu.SemaphoreType.DMA,
            pltpu.SemaphoreType.DMA,
        ],
    )(e, vp, bb2)

    out = pl.pallas_call(
        _out_mlp_body,
        grid=(gn,),
        in_specs=[
            pl.BlockSpec((BN, 128), lambda i: (i, 0)),
            pl.BlockSpec((BN, D), lambda i: (i, 0)),
            pl.BlockSpec((128, 128), lambda i: (0, 0)),
            pl.BlockSpec((1, 128), lambda i: (0, 0)),
            pl.BlockSpec((128, D), lambda i: (0, 0)),
            pl.BlockSpec((1, D), lambda i: (0, 0)),
        ],
        out_specs=pl.BlockSpec((BN, D), lambda i: (i, 0)),
        out_shape=jax.ShapeDtypeStruct((NP, D), jnp.float32),
    )(aggr, xp, Wg1, bg1.reshape(1, 128), Wg2, bg2.reshape(1, D))

    return out[:N]


# speedup vs baseline: 2.4868x; 1.0028x over previous
import jax
import jax.numpy as jnp
from jax import lax
from jax.experimental import pallas as pl
from jax.experimental.pallas import tpu as pltpu
from jax.experimental.pallas import tpu_sc as plsc

NEG = -jnp.inf

_SC_INFO = plsc.get_sparse_core_info()
_NC, _NS = _SC_INFO.num_cores, _SC_INFO.num_subcores
_NW = _NC * _NS

_DIV_MUL = 52429


def _node_pre_body(x_ref, pos8_ref, Wh1_ref, bh1_ref, Wh2_ref, bh2_ref,
                   WfX_ref, W1r_ref, bf1_ref, A_ref, B_ref):
    x = x_ref[...]
    h = jnp.maximum(jnp.dot(x, Wh1_ref[...],
                            preferred_element_type=jnp.float32) + bh1_ref[...], 0.0)
    delta8 = jnp.dot(h, Wh2_ref[...], preferred_element_type=jnp.float32) + bh2_ref[...]
    q8 = delta8 - pos8_ref[...]
    B_ref[...] = jnp.dot(q8, W1r_ref[...], preferred_element_type=jnp.float32)
    A_ref[...] = (jnp.dot(x, WfX_ref[...], preferred_element_type=jnp.float32)
                  + jnp.dot(pos8_ref[...], W1r_ref[...], preferred_element_type=jnp.float32)
                  + bf1_ref[...])


def _k1_body(dstm_ref, cc_ref):
    i = pl.program_id(0)

    @pl.when(i == 0)
    def _():
        cc_ref[...] = jnp.zeros_like(cc_ref)

    b = lax.shift_right_logical(dstm_ref[...] * _DIV_MUL, 24)
    for B in range(_NW):
        eq = (b == B).astype(jnp.float32)
        cc_ref[pl.ds(B, 1), :] += jnp.sum(eq, axis=0, keepdims=True)


def _k2_body(cc_ref, U128_ref, L32_ref, lanebase_ref, bb2_ref):
    cc = cc_ref[...]
    laneprefix = jnp.dot(cc, U128_ref[...], preferred_element_type=jnp.float32)
    totals = jnp.sum(cc, axis=1, keepdims=True)
    ru = jnp.floor((totals + 127.0) * (1.0 / 128.0)) * 128.0
    ab = jnp.dot(L32_ref[...], ru, preferred_element_type=jnp.float32)
    lanebase_ref[...] = ab + laneprefix
    ends = ab + totals
    bb2_ref[...] = jnp.concatenate(
        [ab, ends, jnp.zeros((_NW, 14), jnp.float32)], axis=1).astype(jnp.int32)


def _k3_body(dstm_ref, lanebase_ref, L_ref, slot_ref, carry_ref):
    i = pl.program_id(0)

    @pl.when(i == 0)
    def _():
        carry_ref[...] = jnp.zeros_like(carry_ref)

    b = lax.shift_right_logical(dstm_ref[...] * _DIV_MUL, 24)
    acc = jnp.zeros(slot_ref.shape, jnp.float32)
    L = L_ref[...]
    for B in range(_NW):
        eq = (b == B).astype(jnp.float32)
        p = jnp.dot(L, eq, preferred_element_type=jnp.float32)
        base = carry_ref[pl.ds(B, 1), :] + lanebase_ref[pl.ds(B, 1), :]
        acc = acc + eq * (p + base)
        carry_ref[pl.ds(B, 1), :] += jnp.sum(eq, axis=0, keepdims=True)
    slot_ref[...] = acc.astype(jnp.int32)


def _edge_mlp_body(G_ref, Wf2_ref, bf2_ref, e_ref):
    h = jnp.maximum(G_ref[...], 0.0)
    e_ref[...] = jnp.dot(h, Wf2_ref[...], preferred_element_type=jnp.float32) + bf2_ref[...]


def _out_mlp_body(aggr_ref, x_ref, Wg1_ref, bg1_ref, Wg2_ref, bg2_ref, o_ref):
    a = aggr_ref[...]
    a = jnp.where(jnp.isneginf(a), 0.0, a)
    h = jnp.maximum(jnp.dot(a, Wg1_ref[...],
                            preferred_element_type=jnp.float32) + bg1_ref[...], 0.0)
    o_ref[...] = (jnp.dot(h, Wg2_ref[...], preferred_element_type=jnp.float32)
                  + bg2_ref[...] + x_ref[...])


def _scatter_body(src_hbm, dst_hbm, slot_hbm, vp_hbm,
                  srcb, dstb, slotb, vb2, slotb2, sem_in, sem_sc):
    E = src_hbm.shape[0]
    BT = 2560
    SUB = BT // 128
    nbt = E // BT
    wid = lax.axis_index("s") * _NC + lax.axis_index("c")
    n_my = (nbt - wid + _NW - 1) // _NW

    def bt_body(i, _):
        base = pl.multiple_of((wid + i * _NW) * BT, 128)
        cs = pltpu.async_copy(src_hbm.at[pl.ds(base, BT)], srcb, sem_in)
        cd = pltpu.async_copy(dst_hbm.at[pl.ds(base, BT)], dstb, sem_in)
        cl = pltpu.async_copy(slot_hbm.at[pl.ds(base, BT)], slotb, sem_in)

        @pl.when(i > 0)
        def _():
            for r in range(SUB):
                pltpu.make_async_copy(
                    vb2.at[r], vp_hbm.at[slotb2.at[r]], sem_sc).wait()

        cs.wait()
        cd.wait()
        cl.wait()
        for r in range(SUB):
            for c in range(8):
                src1 = pl.ds(r * 128 + c * 16, 16)
                dst1 = pl.ds(c * 16, 16)
                vb2[r, dst1] = srcb[src1] * 16384 + dstb[src1]
                slotb2[r, dst1] = slotb[src1]
        for r in range(SUB):
            pltpu.async_copy(vb2.at[r], vp_hbm.at[slotb2.at[r]], sem_sc)
        return 0

    lax.fori_loop(0, n_my, bt_body, 0)

    @pl.when(n_my > 0)
    def _():
        for r in range(SUB):
            pltpu.make_async_copy(
                vb2.at[r], vp_hbm.at[slotb2.at[r]], sem_sc).wait()


def _gather_add_body(vp_hbm, A_hbm, B_hbm, G_hbm,
                     vb, sidx2, didx2, rA, rB,
                     sem_in, semA, semB, sem_out):
    EPC = G_hbm.shape[0]
    NPAD = A_hbm.shape[0]
    K = 3
    BT = K * 128
    nbt = EPC // K
    wid = lax.axis_index("s") * _NC + lax.axis_index("c")
    n_my = (nbt - wid + _NW - 1) // _NW

    def bt_body(i, _):
        bt = wid + i * _NW
        base = pl.multiple_of(bt * BT, 128)
        cv = pltpu.async_copy(vp_hbm.at[pl.ds(base, BT)], vb, sem_in)

        @pl.when(i > 0)
        def _():
            btp = wid + (i - 1) * _NW
            pltpu.make_async_copy(
                rA, G_hbm.at[pl.ds(btp * K, K)], sem_out).wait()

        cv.wait()
        for r in range(K):
            for c in range(8):
                src1 = pl.ds(r * 128 + c * 16, 16)
                dst1 = pl.ds(c * 16, 16)
                v = vb[src1]
                sidx2[r, dst1] = jnp.clip(lax.shift_right_logical(v, 14),
                                          0, NPAD - 1)
                didx2[r, dst1] = jnp.clip(v & 16383, 0, NPAD - 1)
        gs = []
        for r in range(K):
            gs.append(pltpu.async_copy(A_hbm.at[sidx2.at[r]], rA.at[r], semA))
            gs.append(pltpu.async_copy(B_hbm.at[didx2.at[r]], rB.at[r], semB))
        for c in gs:
            c.wait()

        def radd(k, _):
            r3 = k >> 7
            kk = k & 127
            for c in range(8):
                sl = pl.ds(c * 16, 16)
                rA[r3, kk, sl] = rA[r3, kk, sl] + rB[r3, kk, sl]
            return 0

        lax.fori_loop(0, BT, radd, 0)
        pltpu.async_copy(rA, G_hbm.at[pl.ds(bt * K, K)], sem_out)
        return 0

    lax.fori_loop(0, n_my, bt_body, 0)

    @pl.when(n_my > 0)
    def _():
        btl = wid + (n_my - 1) * _NW
        pltpu.make_async_copy(
            rA, G_hbm.at[pl.ds(btl * K, K)], sem_out).wait()


def _segmax_body(e_hbm, vp_hbm, bb2_hbm, aggr_hbm,
                 bbv, dv0, dv1, r0, r1, slab, sem, semd):
    NPAD = aggr_hbm.shape[0]
    RW = NPAD // _NW
    SCH = 256
    wid = lax.axis_index("s") * _NC + lax.axis_index("c")
    lo = wid * RW

    pltpu.sync_copy(bb2_hbm, bbv)
    v16 = bbv[wid, pl.ds(0, 16)]
    start = v16[0]
    end = v16[1]

    neg = jnp.full((16,), NEG, jnp.float32)

    def init_slab(r, _):
        for c in range(8):
            slab[r, pl.ds(c * 16, 16)] = neg
        return 0

    lax.fori_loop(0, RW, init_slab, 0)

    rows = [r0, r1]
    dvs = [dv0, dv1]
    nsch = (end - start + SCH - 1) // SCH

    def pair_body(i, _):
        for p in range(2):
            sc = i * 2 + p
            base = pl.multiple_of(start + sc * SCH, 128)

            @pl.when(sc < nsch)
            def _(p=p, base=base):
                pltpu.async_copy(e_hbm.at[pl.ds(base, SCH)], rows[p], sem)
                pltpu.async_copy(vp_hbm.at[pl.ds(base, SCH)], dvs[p], semd)
        for p in range(2):
            sc = i * 2 + p
            base = pl.multiple_of(start + sc * SCH, 128)
            cnt = end - base

            @pl.when(sc < nsch)
            def _(p=p, base=base, cnt=cnt):
                pltpu.make_async_copy(e_hbm.at[pl.ds(base, SCH)], rows[p], sem).wait()
                pltpu.make_async_copy(vp_hbm.at[pl.ds(base, SCH)], dvs[p], semd).wait()

                @pl.when(cnt >= SCH)
                def _():
                    def apply16(g, _):
                        dv = dvs[p][pl.ds(g * 16, 16)] & 16383
                        for j in range(16):
                            k = g * 16 + j
                            ld = dv[j] - lo
                            for c in range(8):
                                sl = pl.ds(c * 16, 16)
                                slab[ld, sl] = jnp.maximum(slab[ld, sl],
                                                           rows[p][k, sl])
                        return 0

                    lax.fori_loop(0, SCH // 16, apply16, 0)

                @pl.when(cnt < SCH)
                def _():
                    def apply16t(g, _):
                        dv = dvs[p][pl.ds(g * 16, 16)] & 16383
                        for j in range(16):
                            k = g * 16 + j

                            @pl.when(k < cnt)
                            def _(j=j, k=k):
                                ld = dv[j] - lo
                                for c in range(8):
                                    sl = pl.ds(c * 16, 16)
                                    slab[ld, sl] = jnp.maximum(slab[ld, sl],
                                                               rows[p][k, sl])
                        return 0

                    lax.fori_loop(0, (cnt + 15) // 16, apply16t, 0)
        return 0

    lax.fori_loop(0, (nsch + 1) // 2, pair_body, 0)
    lo8 = pl.multiple_of(lo, 8)
    pltpu.sync_copy(slab, aggr_hbm.at[pl.ds(lo8, RW)])


def kernel(x, pos, edge_index, Wh1, bh1, Wh2, bh2, Wf1, bf1, Wf2, bf2,
           Wg1, bg1, Wg2, bg2):
    N, D = x.shape
    E = edge_index.shape[1]
    NPM = _NW * 8
    NP = ((N + NPM - 1) // NPM) * NPM
    EP = E + _NW * 128
    ER = 2560

    xp = jnp.pad(x, ((0, NP - N), (0, 0)))
    pos8 = jnp.pad(pos, ((0, NP - N), (0, 5)))
    Wh2p = jnp.pad(Wh2, ((0, 0), (0, 5)))
    bh2p = jnp.pad(bh2, ((0, 5))).reshape(1, 8)
    W1r = jnp.pad(Wf1[:3], ((0, 5), (0, 0)))
    WfX = Wf1[3:]

    BN = 1024
    gn = NP // BN
    A, B = pl.pallas_call(
        _node_pre_body,
        grid=(gn,),
        in_specs=[
            pl.BlockSpec((BN, D), lambda i: (i, 0)),
            pl.BlockSpec((BN, 8), lambda i: (i, 0)),
            pl.BlockSpec((D, 64), lambda i: (0, 0)),
            pl.BlockSpec((1, 64), lambda i: (0, 0)),
            pl.BlockSpec((64, 8), lambda i: (0, 0)),
            pl.BlockSpec((1, 8), lambda i: (0, 0)),
            pl.BlockSpec((D, 128), lambda i: (0, 0)),
            pl.BlockSpec((8, 128), lambda i: (0, 0)),
            pl.BlockSpec((1, 128), lambda i: (0, 0)),
        ],
        out_specs=[
            pl.BlockSpec((BN, 128), lambda i: (i, 0)),
            pl.BlockSpec((BN, 128), lambda i: (i, 0)),
        ],
        out_shape=[
            jax.ShapeDtypeStruct((NP, 128), jnp.float32),
            jax.ShapeDtypeStruct((NP, 128), jnp.float32),
        ],
    )(xp, pos8, Wh1, bh1.reshape(1, 64), Wh2p, bh2p, WfX, W1r,
      bf1.reshape(1, 128))

    src = edge_index[0]
    dst = edge_index[1]
    dstm = jnp.pad(dst, (0, ER * 128 - E), constant_values=NP).reshape(ER, 128)

    RB = 128
    gk = ER // RB
    colcount = pl.pallas_call(
        _k1_body,
        grid=(gk,),
        in_specs=[pl.BlockSpec((RB, 128), lambda i: (i, 0))],
        out_specs=pl.BlockSpec((_NW, 128), lambda i: (0, 0)),
        out_shape=jax.ShapeDtypeStruct((_NW, 128), jnp.float32),
    )(dstm)

    U128 = jnp.triu(jnp.ones((128, 128), jnp.float32), k=1)
    L32 = jnp.tril(jnp.ones((_NW, _NW), jnp.float32), k=-1)
    lanebase, bb2 = pl.pallas_call(
        _k2_body,
        out_shape=[
            jax.ShapeDtypeStruct((_NW, 128), jnp.float32),
            jax.ShapeDtypeStruct((_NW, 16), jnp.int32),
        ],
    )(colcount, U128, L32)

    LRB = jnp.tril(jnp.ones((RB, RB), jnp.float32), k=-1)
    slot = pl.pallas_call(
        _k3_body,
        grid=(gk,),
        in_specs=[
            pl.BlockSpec((RB, 128), lambda i: (i, 0)),
            pl.BlockSpec((_NW, 128), lambda i: (0, 0)),
            pl.BlockSpec((RB, RB), lambda i: (0, 0)),
        ],
        out_specs=pl.BlockSpec((RB, 128), lambda i: (i, 0)),
        out_shape=jax.ShapeDtypeStruct((ER, 128), jnp.int32),
        scratch_shapes=[pltpu.VMEM((_NW, 128), jnp.float32)],
    )(dstm, lanebase, LRB)
    slot_flat = slot.reshape(ER * 128)[:E]

    mesh = plsc.VectorSubcoreMesh(core_axis_name="c", subcore_axis_name="s")

    vp = pl.kernel(
        _scatter_body,
        out_type=jax.ShapeDtypeStruct((EP,), jnp.int32),
        mesh=mesh,
        scratch_types=[
            pltpu.VMEM((2560,), jnp.int32),
            pltpu.VMEM((2560,), jnp.int32),
            pltpu.VMEM((2560,), jnp.int32),
            pltpu.VMEM((20, 128), jnp.int32),
            pltpu.VMEM((20, 128), jnp.int32),
            pltpu.SemaphoreType.DMA,
            pltpu.SemaphoreType.DMA,
        ],
    )(src, dst, slot_flat)

    G3 = pl.kernel(
        _gather_add_body,
        out_type=jax.ShapeDtypeStruct((EP // 128, 128, 128), jnp.float32),
        mesh=mesh,
        scratch_types=[
            pltpu.VMEM((384,), jnp.int32),
            pltpu.VMEM((3, 128), jnp.int32),
            pltpu.VMEM((3, 128), jnp.int32),
            pltpu.VMEM((3, 128, 128), jnp.float32),
            pltpu.VMEM((3, 128, 128), jnp.float32),
            pltpu.SemaphoreType.DMA,
            pltpu.SemaphoreType.DMA,
            pltpu.SemaphoreType.DMA,
            pltpu.SemaphoreType.DMA,
        ],
    )(vp, A, B)
    G = G3.reshape(EP, 128)

    BE = 512
    e = pl.pallas_call(
        _edge_mlp_body,
        grid=(EP // BE,),
        in_specs=[
            pl.BlockSpec((BE, 128), lambda i: (i, 0)),
            pl.BlockSpec((128, 128), lambda i: (0, 0)),
            pl.BlockSpec((1, 128), lambda i: (0, 0)),
        ],
        out_specs=pl.BlockSpec((BE, 128), lambda i: (i, 0)),
        out_shape=jax.ShapeDtypeStruct((EP, 128), jnp.float32),
    )(G, Wf2, bf2.reshape(1, 128))

    RW = NP // _NW
    aggr = pl.kernel(
        _segmax_body,
        out_type=jax.ShapeDtypeStruct((NP, 128), jnp.float32),
        mesh=mesh,
        scratch_types=[
            pltpu.VMEM((_NW, 16), jnp.int32),
            pltpu.VMEM((256,), jnp.int32),
            pltpu.VMEM((256,), jnp.int32),
            pltpu.VMEM((256, 128), jnp.float32),
            pltpu.VMEM((256, 128), jnp.float32),
            pltpu.VMEM((RW, 128), jnp.float32),
            pltpu.SemaphoreType.DMA,
            pltpu.SemaphoreType.DMA,
        ],
    )(e, vp, bb2)

    out = pl.pallas_call(
        _out_mlp_body,
        grid=(gn,),
        in_specs=[
            pl.BlockSpec((BN, 128), lambda i: (i, 0)),
            pl.BlockSpec((BN, D), lambda i: (i, 0)),
            pl.BlockSpec((128, 128), lambda i: (0, 0)),
            pl.BlockSpec((1, 128), lambda i: (0, 0)),
            pl.BlockSpec((128, D), lambda i: (0, 0)),
            pl.BlockSpec((1, D), lambda i: (0, 0)),
        ],
        out_specs=pl.BlockSpec((BN, D), lambda i: (i, 0)),
        out_shape=jax.ShapeDtypeStruct((NP, D), jnp.float32),
    )(aggr, xp, Wg1, bg1.reshape(1, 128), Wg2, bg2.reshape(1, D))

    return out[:N]


# speedup vs baseline: 2.4977x; 1.0044x over previous
import jax
import jax.numpy as jnp
from jax import lax
from jax.experimental import pallas as pl
from jax.experimental.pallas import tpu as pltpu
from jax.experimental.pallas import tpu_sc as plsc

NEG = -jnp.inf

_SC_INFO = plsc.get_sparse_core_info()
_NC, _NS = _SC_INFO.num_cores, _SC_INFO.num_subcores
_NW = _NC * _NS

_DIV_MUL = 52429


def _node_pre_body(x_ref, pos8_ref, Wh1_ref, bh1_ref, Wh2_ref, bh2_ref,
                   WfX_ref, W1r_ref, bf1_ref, A_ref, B_ref):
    x = x_ref[...]
    h = jnp.maximum(jnp.dot(x, Wh1_ref[...],
                            preferred_element_type=jnp.float32) + bh1_ref[...], 0.0)
    delta8 = jnp.dot(h, Wh2_ref[...], preferred_element_type=jnp.float32) + bh2_ref[...]
    q8 = delta8 - pos8_ref[...]
    B_ref[...] = jnp.dot(q8, W1r_ref[...], preferred_element_type=jnp.float32)
    A_ref[...] = (jnp.dot(x, WfX_ref[...], preferred_element_type=jnp.float32)
                  + jnp.dot(pos8_ref[...], W1r_ref[...], preferred_element_type=jnp.float32)
                  + bf1_ref[...])


def _k1_body(dstm_ref, cc_ref):
    b = lax.shift_right_logical(dstm_ref[...] * _DIV_MUL, 24)
    for B in range(_NW):
        eq = (b == B).astype(jnp.float32)
        cc_ref[pl.ds(B, 1), :] = jnp.sum(eq, axis=0, keepdims=True)


def _k2_body(cc_ref, U128_ref, L32_ref, lanebase_ref, bb2_ref):
    cc = cc_ref[...]
    laneprefix = jnp.dot(cc, U128_ref[...], preferred_element_type=jnp.float32)
    totals = jnp.sum(cc, axis=1, keepdims=True)
    ru = jnp.floor((totals + 127.0) * (1.0 / 128.0)) * 128.0
    ab = jnp.dot(L32_ref[...], ru, preferred_element_type=jnp.float32)
    lanebase_ref[...] = ab + laneprefix
    ends = ab + totals
    bb2_ref[...] = jnp.concatenate(
        [ab, ends, jnp.zeros((_NW, 14), jnp.float32)], axis=1).astype(jnp.int32)


def _k3_body(dstm_ref, cc_ref, U128_ref, L32_ref, L_ref,
             slot_ref, bb2_ref, carry_ref, lanebase_ref):
    i = pl.program_id(0)

    @pl.when(i == 0)
    def _():
        carry_ref[...] = jnp.zeros_like(carry_ref)
        cc = cc_ref[...]
        laneprefix = jnp.dot(cc, U128_ref[...], preferred_element_type=jnp.float32)
        totals = jnp.sum(cc, axis=1, keepdims=True)
        ru = jnp.floor((totals + 127.0) * (1.0 / 128.0)) * 128.0
        ab = jnp.dot(L32_ref[...], ru, preferred_element_type=jnp.float32)
        lanebase_ref[...] = ab + laneprefix
        bb2_ref[...] = jnp.concatenate(
            [ab, ab + totals, jnp.zeros((_NW, 14), jnp.float32)],
            axis=1).astype(jnp.int32)

    b = lax.shift_right_logical(dstm_ref[...] * _DIV_MUL, 24)
    acc = jnp.zeros(slot_ref.shape, jnp.float32)
    L = L_ref[...]
    for B in range(_NW):
        eq = (b == B).astype(jnp.float32)
        p = jnp.dot(L, eq, preferred_element_type=jnp.float32)
        base = carry_ref[pl.ds(B, 1), :] + lanebase_ref[pl.ds(B, 1), :]
        acc = acc + eq * (p + base)
        carry_ref[pl.ds(B, 1), :] += jnp.sum(eq, axis=0, keepdims=True)
    slot_ref[...] = acc.astype(jnp.int32)


def _edge_mlp_body(G_ref, Wf2_ref, bf2_ref, e_ref):
    h = jnp.maximum(G_ref[...], 0.0)
    e_ref[...] = jnp.dot(h, Wf2_ref[...], preferred_element_type=jnp.float32) + bf2_ref[...]


def _out_mlp_body(aggr_ref, x_ref, Wg1_ref, bg1_ref, Wg2_ref, bg2_ref, o_ref):
    a = aggr_ref[...]
    a = jnp.where(jnp.isneginf(a), 0.0, a)
    h = jnp.maximum(jnp.dot(a, Wg1_ref[...],
                            preferred_element_type=jnp.float32) + bg1_ref[...], 0.0)
    o_ref[...] = (jnp.dot(h, Wg2_ref[...], preferred_element_type=jnp.float32)
                  + bg2_ref[...] + x_ref[...])


def _scatter_body(src_hbm, dst_hbm, slot_hbm, vp_hbm,
                  srcb, dstb, slotb, vb2, slotb2, sem_in, sem_sc):
    E = src_hbm.shape[0]
    BT = 2560
    SUB = BT // 128
    nbt = E // BT
    wid = lax.axis_index("s") * _NC + lax.axis_index("c")
    n_my = (nbt - wid + _NW - 1) // _NW

    def bt_body(i, _):
        base = pl.multiple_of((wid + i * _NW) * BT, 128)
        cs = pltpu.async_copy(src_hbm.at[pl.ds(base, BT)], srcb, sem_in)
        cd = pltpu.async_copy(dst_hbm.at[pl.ds(base, BT)], dstb, sem_in)
        cl = pltpu.async_copy(slot_hbm.at[pl.ds(base, BT)], slotb, sem_in)

        @pl.when(i > 0)
        def _():
            for r in range(SUB):
                pltpu.make_async_copy(
                    vb2.at[r], vp_hbm.at[slotb2.at[r]], sem_sc).wait()

        cs.wait()
        cd.wait()
        cl.wait()
        for r in range(SUB):
            for c in range(8):
                src1 = pl.ds(r * 128 + c * 16, 16)
                dst1 = pl.ds(c * 16, 16)
                vb2[r, dst1] = srcb[src1] * 16384 + dstb[src1]
                slotb2[r, dst1] = slotb[src1]
        for r in range(SUB):
            pltpu.async_copy(vb2.at[r], vp_hbm.at[slotb2.at[r]], sem_sc)
        return 0

    lax.fori_loop(0, n_my, bt_body, 0)

    @pl.when(n_my > 0)
    def _():
        for r in range(SUB):
            pltpu.make_async_copy(
                vb2.at[r], vp_hbm.at[slotb2.at[r]], sem_sc).wait()


def _gather_add_body(vp_hbm, A_hbm, B_hbm, G_hbm,
                     vb, sidx2, didx2, rA, rB,
                     sem_in, semA, semB, sem_out):
    EPC = G_hbm.shape[0]
    NPAD = A_hbm.shape[0]
    K = 3
    BT = K * 128
    nbt = EPC // K
    wid = lax.axis_index("s") * _NC + lax.axis_index("c")
    n_my = (nbt - wid + _NW - 1) // _NW

    def bt_body(i, _):
        bt = wid + i * _NW
        base = pl.multiple_of(bt * BT, 128)
        cv = pltpu.async_copy(vp_hbm.at[pl.ds(base, BT)], vb, sem_in)

        @pl.when(i > 0)
        def _():
            btp = wid + (i - 1) * _NW
            pltpu.make_async_copy(
                rA, G_hbm.at[pl.ds(btp * K, K)], sem_out).wait()

        cv.wait()
        for r in range(K):
            for c in range(8):
                src1 = pl.ds(r * 128 + c * 16, 16)
                dst1 = pl.ds(c * 16, 16)
                v = vb[src1]
                sidx2[r, dst1] = jnp.clip(lax.shift_right_logical(v, 14),
                                          0, NPAD - 1)
                didx2[r, dst1] = jnp.clip(v & 16383, 0, NPAD - 1)
        gs = []
        for r in range(K):
            gs.append(pltpu.async_copy(A_hbm.at[sidx2.at[r]], rA.at[r], semA))
            gs.append(pltpu.async_copy(B_hbm.at[didx2.at[r]], rB.at[r], semB))
        for c in gs:
            c.wait()

        def radd(k, _):
            r3 = k >> 7
            kk = k & 127
            for c in range(8):
                sl = pl.ds(c * 16, 16)
                rA[r3, kk, sl] = rA[r3, kk, sl] + rB[r3, kk, sl]
            return 0

        lax.fori_loop(0, BT, radd, 0)
        pltpu.async_copy(rA, G_hbm.at[pl.ds(bt * K, K)], sem_out)
        return 0

    lax.fori_loop(0, n_my, bt_body, 0)

    @pl.when(n_my > 0)
    def _():
        btl = wid + (n_my - 1) * _NW
        pltpu.make_async_copy(
            rA, G_hbm.at[pl.ds(btl * K, K)], sem_out).wait()


def _segmax_body(e_hbm, vp_hbm, bb2_hbm, aggr_hbm,
                 bbv, dv0, dv1, r0, r1, slab, sem, semd):
    NPAD = aggr_hbm.shape[0]
    RW = NPAD // _NW
    SCH = 256
    wid = lax.axis_index("s") * _NC + lax.axis_index("c")
    lo = wid * RW

    pltpu.sync_copy(bb2_hbm, bbv)
    v16 = bbv[wid, pl.ds(0, 16)]
    start = v16[0]
    end = v16[1]

    neg = jnp.full((16,), NEG, jnp.float32)

    def init_slab(r, _):
        for c in range(8):
            slab[r, pl.ds(c * 16, 16)] = neg
        return 0

    lax.fori_loop(0, RW, init_slab, 0)

    rows = [r0, r1]
    dvs = [dv0, dv1]
    nsch = (end - start + SCH - 1) // SCH

    def pair_body(i, _):
        for p in range(2):
            sc = i * 2 + p
            base = pl.multiple_of(start + sc * SCH, 128)

            @pl.when(sc < nsch)
            def _(p=p, base=base):
                pltpu.async_copy(e_hbm.at[pl.ds(base, SCH)], rows[p], sem)
                pltpu.async_copy(vp_hbm.at[pl.ds(base, SCH)], dvs[p], semd)
        for p in range(2):
            sc = i * 2 + p
            base = pl.multiple_of(start + sc * SCH, 128)
            cnt = end - base

            @pl.when(sc < nsch)
            def _(p=p, base=base, cnt=cnt):
                pltpu.make_async_copy(e_hbm.at[pl.ds(base, SCH)], rows[p], sem).wait()
                pltpu.make_async_copy(vp_hbm.at[pl.ds(base, SCH)], dvs[p], semd).wait()

                @pl.when(cnt >= SCH)
                def _():
                    def apply16(g, _):
                        dv = dvs[p][pl.ds(g * 16, 16)] & 16383
                        for j in range(16):
                            k = g * 16 + j
                            ld = dv[j] - lo
                            for c in range(8):
                                sl = pl.ds(c * 16, 16)
                                slab[ld, sl] = jnp.maximum(slab[ld, sl],
                                                           rows[p][k, sl])
                        return 0

                    lax.fori_loop(0, SCH // 16, apply16, 0)

                @pl.when(cnt < SCH)
                def _():
                    def apply16t(g, _):
                        dv = dvs[p][pl.ds(g * 16, 16)] & 16383
                        for j in range(16):
                            k = g * 16 + j

                            @pl.when(k < cnt)
                            def _(j=j, k=k):
                                ld = dv[j] - lo
                                for c in range(8):
                                    sl = pl.ds(c * 16, 16)
                                    slab[ld, sl] = jnp.maximum(slab[ld, sl],
                                                               rows[p][k, sl])
                        return 0

                    lax.fori_loop(0, (cnt + 15) // 16, apply16t, 0)
        return 0

    lax.fori_loop(0, (nsch + 1) // 2, pair_body, 0)
    lo8 = pl.multiple_of(lo, 8)
    pltpu.sync_copy(slab, aggr_hbm.at[pl.ds(lo8, RW)])


def kernel(x, pos, edge_index, Wh1, bh1, Wh2, bh2, Wf1, bf1, Wf2, bf2,
           Wg1, bg1, Wg2, bg2):
    N, D = x.shape
    E = edge_index.shape[1]
    NPM = _NW * 8
    NP = ((N + NPM - 1) // NPM) * NPM
    EP = E + _NW * 128
    ER = 2560

    xp = jnp.pad(x, ((0, NP - N), (0, 0)))
    pos8 = jnp.pad(pos, ((0, NP - N), (0, 5)))
    Wh2p = jnp.pad(Wh2, ((0, 0), (0, 5)))
    bh2p = jnp.pad(bh2, ((0, 5))).reshape(1, 8)
    W1r = jnp.pad(Wf1[:3], ((0, 5), (0, 0)))
    WfX = Wf1[3:]

    BN = 1024
    gn = NP // BN
    A, B = pl.pallas_call(
        _node_pre_body,
        grid=(gn,),
        in_specs=[
            pl.BlockSpec((BN, D), lambda i: (i, 0)),
            pl.BlockSpec((BN, 8), lambda i: (i, 0)),
            pl.BlockSpec((D, 64), lambda i: (0, 0)),
            pl.BlockSpec((1, 64), lambda i: (0, 0)),
            pl.BlockSpec((64, 8), lambda i: (0, 0)),
            pl.BlockSpec((1, 8), lambda i: (0, 0)),
            pl.BlockSpec((D, 128), lambda i: (0, 0)),
            pl.BlockSpec((8, 128), lambda i: (0, 0)),
            pl.BlockSpec((1, 128), lambda i: (0, 0)),
        ],
        out_specs=[
            pl.BlockSpec((BN, 128), lambda i: (i, 0)),
            pl.BlockSpec((BN, 128), lambda i: (i, 0)),
        ],
        out_shape=[
            jax.ShapeDtypeStruct((NP, 128), jnp.float32),
            jax.ShapeDtypeStruct((NP, 128), jnp.float32),
        ],
    )(xp, pos8, Wh1, bh1.reshape(1, 64), Wh2p, bh2p, WfX, W1r,
      bf1.reshape(1, 128))

    src = edge_index[0]
    dst = edge_index[1]
    dstm = jnp.pad(dst, (0, ER * 128 - E), constant_values=NP).reshape(ER, 128)

    colcount = pl.pallas_call(
        _k1_body,
        out_shape=jax.ShapeDtypeStruct((_NW, 128), jnp.float32),
    )(dstm)

    RB = 128
    gk = ER // RB
    U128 = jnp.triu(jnp.ones((128, 128), jnp.float32), k=1)
    L32 = jnp.tril(jnp.ones((_NW, _NW), jnp.float32), k=-1)
    LRB = jnp.tril(jnp.ones((RB, RB), jnp.float32), k=-1)
    slot, bb2 = pl.pallas_call(
        _k3_body,
        grid=(gk,),
        in_specs=[
            pl.BlockSpec((RB, 128), lambda i: (i, 0)),
            pl.BlockSpec((_NW, 128), lambda i: (0, 0)),
            pl.BlockSpec((128, 128), lambda i: (0, 0)),
            pl.BlockSpec((_NW, _NW), lambda i: (0, 0)),
            pl.BlockSpec((RB, RB), lambda i: (0, 0)),
        ],
        out_specs=[
            pl.BlockSpec((RB, 128), lambda i: (i, 0)),
            pl.BlockSpec((_NW, 16), lambda i: (0, 0)),
        ],
        out_shape=[
            jax.ShapeDtypeStruct((ER, 128), jnp.int32),
            jax.ShapeDtypeStruct((_NW, 16), jnp.int32),
        ],
        scratch_shapes=[
            pltpu.VMEM((_NW, 128), jnp.float32),
            pltpu.VMEM((_NW, 128), jnp.float32),
        ],
    )(dstm, colcount, U128, L32, LRB)
    slot_flat = slot.reshape(ER * 128)[:E]

    mesh = plsc.VectorSubcoreMesh(core_axis_name="c", subcore_axis_name="s")

    vp = pl.kernel(
        _scatter_body,
        out_type=jax.ShapeDtypeStruct((EP,), jnp.int32),
        mesh=mesh,
        scratch_types=[
            pltpu.VMEM((2560,), jnp.int32),
            pltpu.VMEM((2560,), jnp.int32),
            pltpu.VMEM((2560,), jnp.int32),
            pltpu.VMEM((20, 128), jnp.int32),
            pltpu.VMEM((20, 128), jnp.int32),
            pltpu.SemaphoreType.DMA,
            pltpu.SemaphoreType.DMA,
        ],
    )(src, dst, slot_flat)

    G3 = pl.kernel(
        _gather_add_body,
        out_type=jax.ShapeDtypeStruct((EP // 128, 128, 128), jnp.float32),
        mesh=mesh,
        scratch_types=[
            pltpu.VMEM((384,), jnp.int32),
            pltpu.VMEM((3, 128), jnp.int32),
            pltpu.VMEM((3, 128), jnp.int32),
            pltpu.VMEM((3, 128, 128), jnp.float32),
            pltpu.VMEM((3, 128, 128), jnp.float32),
            pltpu.SemaphoreType.DMA,
            pltpu.SemaphoreType.DMA,
            pltpu.SemaphoreType.DMA,
            pltpu.SemaphoreType.DMA,
        ],
    )(vp, A, B)
    G = G3.reshape(EP, 128)

    BE = 512
    e = pl.pallas_call(
        _edge_mlp_body,
        grid=(EP // BE,),
        in_specs=[
            pl.BlockSpec((BE, 128), lambda i: (i, 0)),
            pl.BlockSpec((128, 128), lambda i: (0, 0)),
            pl.BlockSpec((1, 128), lambda i: (0, 0)),
        ],
        out_specs=pl.BlockSpec((BE, 128), lambda i: (i, 0)),
        out_shape=jax.ShapeDtypeStruct((EP, 128), jnp.float32),
    )(G, Wf2, bf2.reshape(1, 128))

    RW = NP // _NW
    aggr = pl.kernel(
        _segmax_body,
        out_type=jax.ShapeDtypeStruct((NP, 128), jnp.float32),
        mesh=mesh,
        scratch_types=[
            pltpu.VMEM((_NW, 16), jnp.int32),
            pltpu.VMEM((256,), jnp.int32),
            pltpu.VMEM((256,), jnp.int32),
            pltpu.VMEM((256, 128), jnp.float32),
            pltpu.VMEM((256, 128), jnp.float32),
            pltpu.VMEM((RW, 128), jnp.float32),
            pltpu.SemaphoreType.DMA,
            pltpu.SemaphoreType.DMA,
        ],
    )(e, vp, bb2)

    out = pl.pallas_call(
        _out_mlp_body,
        grid=(gn,),
        in_specs=[
            pl.BlockSpec((BN, 128), lambda i: (i, 0)),
            pl.BlockSpec((BN, D), lambda i: (i, 0)),
            pl.BlockSpec((128, 128), lambda i: (0, 0)),
            pl.BlockSpec((1, 128), lambda i: (0, 0)),
            pl.BlockSpec((128, D), lambda i: (0, 0)),
            pl.BlockSpec((1, D), lambda i: (0, 0)),
        ],
        out_specs=pl.BlockSpec((BN, D), lambda i: (i, 0)),
        out_shape=jax.ShapeDtypeStruct((NP, D), jnp.float32),
    )(aggr, xp, Wg1, bg1.reshape(1, 128), Wg2, bg2.reshape(1, D))

    return out[:N]


# speedup vs baseline: 2.8834x; 1.1544x over previous
import jax
import jax.numpy as jnp
from jax import lax
from jax.experimental import pallas as pl
from jax.experimental.pallas import tpu as pltpu
from jax.experimental.pallas import tpu_sc as plsc

NEG = -jnp.inf

_SC_INFO = plsc.get_sparse_core_info()
_NC, _NS = _SC_INFO.num_cores, _SC_INFO.num_subcores
_NW = _NC * _NS

_DIV_MUL = 52429


def _node_pre_body(x_ref, pos8_ref, Wh1_ref, bh1_ref, Wh2_ref, bh2_ref,
                   WfX_ref, W1r_ref, bf1_ref, A_ref, B_ref):
    x = x_ref[...]
    h = jnp.maximum(jnp.dot(x, Wh1_ref[...],
                            preferred_element_type=jnp.float32) + bh1_ref[...], 0.0)
    delta8 = jnp.dot(h, Wh2_ref[...], preferred_element_type=jnp.float32) + bh2_ref[...]
    q8 = delta8 - pos8_ref[...]
    B_ref[...] = jnp.dot(q8, W1r_ref[...], preferred_element_type=jnp.float32)
    A_ref[...] = (jnp.dot(x, WfX_ref[...], preferred_element_type=jnp.float32)
                  + jnp.dot(pos8_ref[...], W1r_ref[...], preferred_element_type=jnp.float32)
                  + bf1_ref[...])


def _k1_body(dstm_ref, cc_ref):
    b = lax.shift_right_logical(dstm_ref[...] * _DIV_MUL, 24)
    for B in range(_NW):
        eq = (b == B).astype(jnp.float32)
        cc_ref[pl.ds(B, 1), :] = jnp.sum(eq, axis=0, keepdims=True)


def _k2_body(cc_ref, U128_ref, L32_ref, lanebase_ref, bb2_ref):
    cc = cc_ref[...]
    laneprefix = jnp.dot(cc, U128_ref[...], preferred_element_type=jnp.float32)
    totals = jnp.sum(cc, axis=1, keepdims=True)
    ru = jnp.floor((totals + 127.0) * (1.0 / 128.0)) * 128.0
    ab = jnp.dot(L32_ref[...], ru, preferred_element_type=jnp.float32)
    lanebase_ref[...] = ab + laneprefix
    ends = ab + totals
    bb2_ref[...] = jnp.concatenate(
        [ab, ends, jnp.zeros((_NW, 14), jnp.float32)], axis=1).astype(jnp.int32)


def _k3_body(dstm_ref, cc_ref, U128_ref, L32_ref, L_ref,
             slot_ref, bb2_ref, carry_ref, lanebase_ref):
    i = pl.program_id(0)

    @pl.when(i == 0)
    def _():
        carry_ref[...] = jnp.zeros_like(carry_ref)
        cc = cc_ref[...]
        laneprefix = jnp.dot(cc, U128_ref[...], preferred_element_type=jnp.float32)
        totals = jnp.sum(cc, axis=1, keepdims=True)
        ru = jnp.floor((totals + 127.0) * (1.0 / 128.0)) * 128.0
        ab = jnp.dot(L32_ref[...], ru, preferred_element_type=jnp.float32)
        lanebase_ref[...] = ab + laneprefix
        bb2_ref[...] = jnp.concatenate(
            [ab, ab + totals, jnp.zeros((_NW, 14), jnp.float32)],
            axis=1).astype(jnp.int32)

    b = lax.shift_right_logical(dstm_ref[...] * _DIV_MUL, 24)
    acc = jnp.zeros(slot_ref.shape, jnp.float32)
    L = L_ref[...]
    for B in range(_NW):
        eq = (b == B).astype(jnp.float32)
        p = jnp.dot(L, eq, preferred_element_type=jnp.float32)
        base = carry_ref[pl.ds(B, 1), :] + lanebase_ref[pl.ds(B, 1), :]
        acc = acc + eq * (p + base)
        carry_ref[pl.ds(B, 1), :] += jnp.sum(eq, axis=0, keepdims=True)
    slot_ref[...] = acc.astype(jnp.int32)


def _edge_mlp_body(G_ref, Wf2_ref, bf2_ref, e_ref):
    h = jnp.maximum(G_ref[...], 0.0)
    e_ref[...] = jnp.dot(h, Wf2_ref[...], preferred_element_type=jnp.float32) + bf2_ref[...]


def _out_mlp_body(aggr_ref, x_ref, Wg1_ref, bg1_ref, Wg2_ref, bg2_ref, o_ref):
    a = aggr_ref[...]
    a = jnp.where(jnp.isneginf(a), 0.0, a)
    h = jnp.maximum(jnp.dot(a, Wg1_ref[...],
                            preferred_element_type=jnp.float32) + bg1_ref[...], 0.0)
    o_ref[...] = (jnp.dot(h, Wg2_ref[...], preferred_element_type=jnp.float32)
                  + bg2_ref[...] + x_ref[...])


def _scatter_body(src_hbm, dst_hbm, slot_hbm, vp_hbm,
                  srcb, dstb, slotb, vb2, slotb2, sem_in, sem_sc):
    E = src_hbm.shape[0]
    BT = 2560
    SUB = BT // 128
    nbt = E // BT
    wid = lax.axis_index("s") * _NC + lax.axis_index("c")
    n_my = (nbt - wid + _NW - 1) // _NW

    def bt_body(i, _):
        base = pl.multiple_of((wid + i * _NW) * BT, 128)
        cs = pltpu.async_copy(src_hbm.at[pl.ds(base, BT)], srcb, sem_in)
        cd = pltpu.async_copy(dst_hbm.at[pl.ds(base, BT)], dstb, sem_in)
        cl = pltpu.async_copy(slot_hbm.at[pl.ds(base, BT)], slotb, sem_in)

        @pl.when(i > 0)
        def _():
            for r in range(SUB):
                pltpu.make_async_copy(
                    vb2.at[r], vp_hbm.at[slotb2.at[r]], sem_sc).wait()

        cs.wait()
        cd.wait()
        cl.wait()
        for r in range(SUB):
            for c in range(8):
                src1 = pl.ds(r * 128 + c * 16, 16)
                dst1 = pl.ds(c * 16, 16)
                vb2[r, dst1] = srcb[src1] * 16384 + dstb[src1]
                slotb2[r, dst1] = slotb[src1]
        for r in range(SUB):
            pltpu.async_copy(vb2.at[r], vp_hbm.at[slotb2.at[r]], sem_sc)
        return 0

    lax.fori_loop(0, n_my, bt_body, 0)

    @pl.when(n_my > 0)
    def _():
        for r in range(SUB):
            pltpu.make_async_copy(
                vb2.at[r], vp_hbm.at[slotb2.at[r]], sem_sc).wait()


def _gather_add_body(vp_hbm, A_hbm, B_hbm, G_hbm,
                     vb, sidx2, didx2, rA, rB,
                     sem_in, semA, semB, sem_out):
    EPC = G_hbm.shape[0]
    NPAD = A_hbm.shape[0]
    K = 3
    BT = K * 128
    nbt = EPC // K
    wid = lax.axis_index("s") * _NC + lax.axis_index("c")
    n_my = (nbt - wid + _NW - 1) // _NW

    def bt_body(i, _):
        bt = wid + i * _NW
        base = pl.multiple_of(bt * BT, 128)
        cv = pltpu.async_copy(vp_hbm.at[pl.ds(base, BT)], vb, sem_in)

        @pl.when(i > 0)
        def _():
            btp = wid + (i - 1) * _NW
            pltpu.make_async_copy(
                rA, G_hbm.at[pl.ds(btp * K, K)], sem_out).wait()

        cv.wait()
        for r in range(K):
            for c in range(8):
                src1 = pl.ds(r * 128 + c * 16, 16)
                dst1 = pl.ds(c * 16, 16)
                v = vb[src1]
                sidx2[r, dst1] = jnp.clip(lax.shift_right_logical(v, 14),
                                          0, NPAD - 1)
                didx2[r, dst1] = jnp.clip(v & 16383, 0, NPAD - 1)
        gs = []
        for r in range(K):
            gs.append(pltpu.async_copy(A_hbm.at[sidx2.at[r]], rA.at[r], semA))
            gs.append(pltpu.async_copy(B_hbm.at[didx2.at[r]], rB.at[r], semB))
        for c in gs:
            c.wait()

        def radd(k, _):
            r3 = k >> 7
            kk = k & 127
            for c in range(8):
                sl = pl.ds(c * 16, 16)
                rA[r3, kk, sl] = rA[r3, kk, sl] + rB[r3, kk, sl]
            return 0

        lax.fori_loop(0, BT, radd, 0)
        pltpu.async_copy(rA, G_hbm.at[pl.ds(bt * K, K)], sem_out)
        return 0

    lax.fori_loop(0, n_my, bt_body, 0)

    @pl.when(n_my > 0)
    def _():
        btl = wid + (n_my - 1) * _NW
        pltpu.make_async_copy(
            rA, G_hbm.at[pl.ds(btl * K, K)], sem_out).wait()


def _segmax_body(e_hbm, vp_hbm, bb2_hbm, aggr_hbm,
                 bbv, dv0, dv1, r0, r1, slab, sem, semd):
    NPAD = aggr_hbm.shape[0]
    RW = NPAD // _NW
    SCH = 256
    wid = lax.axis_index("s") * _NC + lax.axis_index("c")
    lo = wid * RW

    pltpu.sync_copy(bb2_hbm, bbv)
    v16 = bbv[wid, pl.ds(0, 16)]
    start = v16[0]
    end = v16[1]

    neg = jnp.full((16,), NEG, jnp.float32)

    def init_slab(r, _):
        for c in range(8):
            slab[r, pl.ds(c * 16, 16)] = neg
        return 0

    lax.fori_loop(0, RW, init_slab, 0)

    rows = [r0, r1]
    dvs = [dv0, dv1]
    nsch = (end - start + SCH - 1) // SCH

    def pair_body(i, _):
        for p in range(2):
            sc = i * 2 + p
            base = pl.multiple_of(start + sc * SCH, 128)

            @pl.when(sc < nsch)
            def _(p=p, base=base):
                pltpu.async_copy(e_hbm.at[pl.ds(base, SCH)], rows[p], sem)
                pltpu.async_copy(vp_hbm.at[pl.ds(base, SCH)], dvs[p], semd)
        for p in range(2):
            sc = i * 2 + p
            base = pl.multiple_of(start + sc * SCH, 128)
            cnt = end - base

            @pl.when(sc < nsch)
            def _(p=p, base=base, cnt=cnt):
                pltpu.make_async_copy(e_hbm.at[pl.ds(base, SCH)], rows[p], sem).wait()
                pltpu.make_async_copy(vp_hbm.at[pl.ds(base, SCH)], dvs[p], semd).wait()

                @pl.when(cnt >= SCH)
                def _():
                    def apply16(g, _):
                        dv = dvs[p][pl.ds(g * 16, 16)] & 16383
                        for j in range(16):
                            k = g * 16 + j
                            ld = dv[j] - lo
                            for c in range(8):
                                sl = pl.ds(c * 16, 16)
                                slab[ld, sl] = jnp.maximum(slab[ld, sl],
                                                           rows[p][k, sl])
                        return 0

                    lax.fori_loop(0, SCH // 16, apply16, 0)

                @pl.when(cnt < SCH)
                def _():
                    def apply16t(g, _):
                        dv = dvs[p][pl.ds(g * 16, 16)] & 16383
                        for j in range(16):
                            k = g * 16 + j

                            @pl.when(k < cnt)
                            def _(j=j, k=k):
                                ld = dv[j] - lo
                                for c in range(8):
                                    sl = pl.ds(c * 16, 16)
                                    slab[ld, sl] = jnp.maximum(slab[ld, sl],
                                                               rows[p][k, sl])
                        return 0

                    lax.fori_loop(0, (cnt + 15) // 16, apply16t, 0)
        return 0

    lax.fori_loop(0, (nsch + 1) // 2, pair_body, 0)
    lo8 = pl.multiple_of(lo, 8)
    pltpu.sync_copy(slab, aggr_hbm.at[pl.ds(lo8, RW)])


def kernel(x, pos, edge_index, Wh1, bh1, Wh2, bh2, Wf1, bf1, Wf2, bf2,
           Wg1, bg1, Wg2, bg2):
    N, D = x.shape
    E = edge_index.shape[1]
    NPM = _NW * 8
    NP = ((N + NPM - 1) // NPM) * NPM
    EP = E + _NW * 128
    ER = 2560

    xp = jnp.pad(x, ((0, NP - N), (0, 0)))
    pos8 = jnp.pad(pos, ((0, NP - N), (0, 5)))
    Wh2p = jnp.pad(Wh2, ((0, 0), (0, 5)))
    bh2p = jnp.pad(bh2, ((0, 5))).reshape(1, 8)
    W1r = jnp.pad(Wf1[:3], ((0, 5), (0, 0)))
    WfX = Wf1[3:]

    BN = 1024
    gn = NP // BN
    A, B = pl.pallas_call(
        _node_pre_body,
        grid=(gn,),
        in_specs=[
            pl.BlockSpec((BN, D), lambda i: (i, 0)),
            pl.BlockSpec((BN, 8), lambda i: (i, 0)),
            pl.BlockSpec((D, 64), lambda i: (0, 0)),
            pl.BlockSpec((1, 64), lambda i: (0, 0)),
            pl.BlockSpec((64, 8), lambda i: (0, 0)),
            pl.BlockSpec((1, 8), lambda i: (0, 0)),
            pl.BlockSpec((D, 128), lambda i: (0, 0)),
            pl.BlockSpec((8, 128), lambda i: (0, 0)),
            pl.BlockSpec((1, 128), lambda i: (0, 0)),
        ],
        out_specs=[
            pl.BlockSpec((BN, 128), lambda i: (i, 0)),
            pl.BlockSpec((BN, 128), lambda i: (i, 0)),
        ],
        out_shape=[
            jax.ShapeDtypeStruct((NP, 128), jnp.float32),
            jax.ShapeDtypeStruct((NP, 128), jnp.float32),
        ],
    )(xp, pos8, Wh1, bh1.reshape(1, 64), Wh2p, bh2p, WfX, W1r,
      bf1.reshape(1, 128))

    src = edge_index[0]
    dst = edge_index[1]
    dstm = jnp.pad(dst, (0, ER * 128 - E), constant_values=NP).reshape(ER, 128)

    colcount = pl.pallas_call(
        _k1_body,
        out_shape=jax.ShapeDtypeStruct((_NW, 128), jnp.float32),
    )(dstm)

    RB = 128
    gk = ER // RB
    U128 = jnp.triu(jnp.ones((128, 128), jnp.float32), k=1)
    L32 = jnp.tril(jnp.ones((_NW, _NW), jnp.float32), k=-1)
    LRB = jnp.tril(jnp.ones((RB, RB), jnp.float32), k=-1)
    slot, bb2 = pl.pallas_call(
        _k3_body,
        grid=(gk,),
        in_specs=[
            pl.BlockSpec((RB, 128), lambda i: (i, 0)),
            pl.BlockSpec((_NW, 128), lambda i: (0, 0)),
            pl.BlockSpec((128, 128), lambda i: (0, 0)),
            pl.BlockSpec((_NW, _NW), lambda i: (0, 0)),
            pl.BlockSpec((RB, RB), lambda i: (0, 0)),
        ],
        out_specs=[
            pl.BlockSpec((RB, 128), lambda i: (i, 0)),
            pl.BlockSpec((_NW, 16), lambda i: (0, 0)),
        ],
        out_shape=[
            jax.ShapeDtypeStruct((ER, 128), jnp.int32),
            jax.ShapeDtypeStruct((_NW, 16), jnp.int32),
        ],
        scratch_shapes=[
            pltpu.VMEM((_NW, 128), jnp.float32),
            pltpu.VMEM((_NW, 128), jnp.float32),
        ],
    )(dstm, colcount, U128, L32, LRB)
    slot_flat = slot.reshape(ER * 128)[:E]

    mesh = plsc.VectorSubcoreMesh(core_axis_name="c", subcore_axis_name="s")

    vp = pl.kernel(
        _scatter_body,
        out_type=jax.ShapeDtypeStruct((EP,), jnp.int32),
        mesh=mesh,
        scratch_types=[
            pltpu.VMEM((2560,), jnp.int32),
            pltpu.VMEM((2560,), jnp.int32),
            pltpu.VMEM((2560,), jnp.int32),
            pltpu.VMEM((20, 128), jnp.int32),
            pltpu.VMEM((20, 128), jnp.int32),
            pltpu.SemaphoreType.DMA,
            pltpu.SemaphoreType.DMA,
        ],
    )(src, dst, slot_flat)

    G3 = pl.kernel(
        _gather_add_body,
        out_type=jax.ShapeDtypeStruct((EP // 128, 128, 128), jnp.float32),
        mesh=mesh,
        scratch_types=[
            pltpu.VMEM((384,), jnp.int32),
            pltpu.VMEM((3, 128), jnp.int32),
            pltpu.VMEM((3, 128), jnp.int32),
            pltpu.VMEM((3, 128, 128), jnp.float32),
            pltpu.VMEM((3, 128, 128), jnp.float32),
            pltpu.SemaphoreType.DMA,
            pltpu.SemaphoreType.DMA,
            pltpu.SemaphoreType.DMA,
            pltpu.SemaphoreType.DMA,
        ],
    )(vp, A, B)
    G = G3.reshape(EP, 128)

    BE = 1536
    e = pl.pallas_call(
        _edge_mlp_body,
        grid=(EP // BE,),
        in_specs=[
            pl.BlockSpec((BE, 128), lambda i: (i, 0)),
            pl.BlockSpec((128, 128), lambda i: (0, 0)),
            pl.BlockSpec((1, 128), lambda i: (0, 0)),
        ],
        out_specs=pl.BlockSpec((BE, 128), lambda i: (i, 0)),
        out_shape=jax.ShapeDtypeStruct((EP, 128), jnp.float32),
    )(G, Wf2, bf2.reshape(1, 128))

    RW = NP // _NW
    aggr = pl.kernel(
        _segmax_body,
        out_type=jax.ShapeDtypeStruct((NP, 128), jnp.float32),
        mesh=mesh,
        scratch_types=[
            pltpu.VMEM((_NW, 16), jnp.int32),
            pltpu.VMEM((256,), jnp.int32),
            pltpu.VMEM((256,), jnp.int32),
            pltpu.VMEM((256, 128), jnp.float32),
            pltpu.VMEM((256, 128), jnp.float32),
            pltpu.VMEM((RW, 128), jnp.float32),
            pltpu.SemaphoreType.DMA,
            pltpu.SemaphoreType.DMA,
        ],
    )(e, vp, bb2)

    out = pl.pallas_call(
        _out_mlp_body,
        grid=(gn,),
        in_specs=[
            pl.BlockSpec((BN, 128), lambda i: (i, 0)),
            pl.BlockSpec((BN, D), lambda i: (i, 0)),
            pl.BlockSpec((128, 128), lambda i: (0, 0)),
            pl.BlockSpec((1, 128), lambda i: (0, 0)),
            pl.BlockSpec((128, D), lambda i: (0, 0)),
            pl.BlockSpec((1, D), lambda i: (0, 0)),
        ],
        out_specs=pl.BlockSpec((BN, D), lambda i: (i, 0)),
        out_shape=jax.ShapeDtypeStruct((NP, D), jnp.float32),
    )(aggr, xp, Wg1, bg1.reshape(1, 128), Wg2, bg2.reshape(1, D))

    return out[:N]
